# Initial kernel scaffold; baseline (speedup 1.0000x reference)
import jax
import jax.numpy as jnp
from jax.experimental import pallas as pl


def kernel(discrete_x, continous_x, edge_index, edge_attr, churn_date, t, Wc, bc, We1, be1, ge1, bte1, We2, be2, ge2, bte2, Wg0, bg0, Wgc1, bgc1, Wgc2, bgc2, Wr1, br1, gr1, btr1, Wr2, br2, gr2, btr2, wcr, bcr, Wf, bf, Ws0, bs0, Wa1, as1, ad1, ae1, bA1, Wa2, as2, ad2, ae2, bA2, Wy0, by0, Wy1, by1, Wp0a, bp0a, Wp0b, bp0b, Wp1a, bp1a, Wp1b, bp1b, WpT, bpT):
    raise NotImplementedError("write your pallas kernel here")


# speedup vs baseline: 21.4096x; 21.4096x over previous
import functools

import jax
import jax.numpy as jnp
from jax import lax
from jax.experimental import pallas as pl
from jax.experimental.pallas import tpu as pltpu
from jax.experimental.pallas import tpu_sc as plsc

N = 50000
E = 800000
B = 10000
H = 64
NE = 16
NH1 = 90

NP = 50176
RB = 3136
NG = NP // RB
TRASH = N

EP = 802816
SLICE = 25088
CH = 196
EB = 8192
EG = EP // EB

def _sc_kernel(**kw):
    def deco(fn):
        cache = {}

        def call(*args):
            if 'k' not in cache:
                mesh = plsc.VectorSubcoreMesh(
                    core_axis_name="c", subcore_axis_name="s",
                    num_cores=2, num_subcores=16)
                cache['k'] = pl.kernel(
                    fn, mesh=mesh,
                    compiler_params=pltpu.CompilerParams(
                        use_tc_tiling_on_sc=False,
                        needs_layout_passes=False),
                    **kw)
            return cache['k'](*args)

        return call

    return deco


def _mm(a, b):
    return jnp.dot(a, b, preferred_element_type=jnp.float32)


@_sc_kernel(
    out_type=jax.ShapeDtypeStruct((2, NP, 8), jnp.float32),
    scratch_types=[
        pltpu.VMEM((CH, 128), jnp.int32),
        pltpu.VMEM((128, 8), jnp.float32),
        pltpu.VMEM_SHARED((NP, 8), jnp.float32),
    ],
)
def _sc_degree(dst3, zeros8, ones8, out, idx_v, ones_v, acc_sh):
    c = lax.axis_index("c")
    s = lax.axis_index("s")
    wid = s * 2 + c
    r0 = s * RB
    pltpu.sync_copy(zeros8.at[pl.ds(r0, RB)], acc_sh.at[pl.ds(r0, RB)])
    pltpu.sync_copy(ones8, ones_v)
    pltpu.sync_copy(dst3.at[wid], idx_v)
    plsc.subcore_barrier()

    def body(j, carry):
        pltpu.sync_copy(ones_v, acc_sh.at[idx_v.at[j]], add=True)
        return carry

    lax.fori_loop(0, CH, body, 0)
    plsc.subcore_barrier()
    pltpu.sync_copy(acc_sh.at[pl.ds(r0, RB)], out.at[c, pl.ds(r0, RB)])


@_sc_kernel(
    out_type=jax.ShapeDtypeStruct((2, NP, 8), jnp.float32),
    scratch_types=[
        pltpu.VMEM((CH, 128), jnp.int32),
        pltpu.VMEM((4, 128, 8), jnp.float32),
        pltpu.VMEM_SHARED((NP, 8), jnp.float32),
        pltpu.SemaphoreType.DMA((4,)),
    ],
)
def _sc_den(dst3, ex8, zeros8, out, idx_v, bufs, acc_sh, sems):
    c = lax.axis_index("c")
    s = lax.axis_index("s")
    wid = s * 2 + c
    base = wid * SLICE
    r0 = s * RB
    pltpu.sync_copy(zeros8.at[pl.ds(r0, RB)], acc_sh.at[pl.ds(r0, RB)])
    pltpu.sync_copy(dst3.at[wid], idx_v)
    plsc.subcore_barrier()

    def ld(j, b):
        return pltpu.make_async_copy(
            ex8.at[pl.ds(base + j * 128, 128)], bufs.at[b], sems.at[b])

    for b in range(4):
        ld(b, b).start()

    def grp(g, carry):
        for b in range(4):
            j = g * 4 + b
            ld(j, b).wait()
            pltpu.sync_copy(bufs.at[b], acc_sh.at[idx_v.at[j]], add=True)
            nxt = j + 4

            @pl.when(nxt < CH)
            def _():
                ld(nxt, b).start()
        return carry

    lax.fori_loop(0, CH // 4, grp, 0)
    plsc.subcore_barrier()
    pltpu.sync_copy(acc_sh.at[pl.ds(r0, RB)], out.at[c, pl.ds(r0, RB)])


@_sc_kernel(
    out_type=jax.ShapeDtypeStruct((4, NP, 16), jnp.float32),
    scratch_types=[
        pltpu.VMEM((CH, 128), jnp.int32),
        pltpu.VMEM((CH, 128), jnp.int32),
        pltpu.VMEM((4, 128, 16), jnp.float32),
        pltpu.VMEM_SHARED((NP, 16), jnp.float32),
        pltpu.SemaphoreType.DMA((4,)),
    ],
)
def _sc_gcn(tab4, src3, dst3, zeros16, out, idxs_v, idxd_v, bufs, acc_sh, sems):
    c = lax.axis_index("c")
    s = lax.axis_index("s")
    r0 = s * RB
    for p in range(2):
        cg = c * 2 + p
        pltpu.sync_copy(zeros16.at[pl.ds(r0, RB)], acc_sh.at[pl.ds(r0, RB)])
        plsc.subcore_barrier()
        for half in range(2):
            sl = s * 2 + half
            pltpu.sync_copy(src3.at[sl], idxs_v)
            pltpu.sync_copy(dst3.at[sl], idxd_v)

            def gat(j, b):
                return pltpu.make_async_copy(
                    tab4.at[cg].at[idxs_v.at[j]], bufs.at[b], sems.at[b])

            for b in range(4):
                gat(b, b).start()

            def grp(g, carry):
                for b in range(4):
                    j = g * 4 + b
                    gat(j, b).wait()
                    pltpu.sync_copy(bufs.at[b], acc_sh.at[idxd_v.at[j]],
                                    add=True)
                    nxt = j + 4

                    @pl.when(nxt < CH)
                    def _():
                        gat(nxt, b).start()
                return carry

            lax.fori_loop(0, CH // 4, grp, 0)
        plsc.subcore_barrier()
        pltpu.sync_copy(acc_sh.at[pl.ds(r0, RB)], out.at[cg, pl.ds(r0, RB)])
        plsc.subcore_barrier()


@_sc_kernel(
    out_type=jax.ShapeDtypeStruct((2, EP, 16), jnp.float32),
    scratch_types=[
        pltpu.VMEM((CH, 128), jnp.int32),
        pltpu.VMEM((4, 128, 16), jnp.float32),
        pltpu.SemaphoreType.DMA((4,)),
    ],
)
def _sc_dualgather(tab, idx_comb, out, idx_v, bufs, sems):
    c = lax.axis_index("c")
    s = lax.axis_index("s")
    for half in range(2):
        sl = s * 2 + half
        base = sl * SLICE
        pltpu.sync_copy(idx_comb.at[c, sl], idx_v)

        def gat(j, b):
            return pltpu.make_async_copy(
                tab.at[idx_v.at[j]], bufs.at[b], sems.at[b])

        for b in range(4):
            gat(b, b).start()

        def grp(g, carry):
            for b in range(4):
                j = g * 4 + b
                gat(j, b).wait()
                pltpu.sync_copy(
                    bufs.at[b], out.at[c, pl.ds(base + j * 128, 128)])
                nxt = j + 4

                @pl.when(nxt < CH)
                def _():
                    gat(nxt, b).start()
            return carry

        lax.fori_loop(0, CH // 4, grp, 0)


QCH = CH // 2


@_sc_kernel(
    out_type=jax.ShapeDtypeStruct((2, 4, NP, 16), jnp.float32),
    scratch_types=[
        pltpu.VMEM((QCH, 128), jnp.int32),
        pltpu.VMEM((QCH, 128), jnp.int32),
        pltpu.VMEM((QCH, 128), jnp.float32),
        pltpu.VMEM((2, 128, 16), jnp.float32),
        pltpu.VMEM_SHARED((NP, 16), jnp.float32),
        pltpu.SemaphoreType.DMA((2,)),
    ],
)
def _sc_egat(tabw, src3, dst3, ex4, zeros16, out,
             idxs_v, idxd_v, ex_v, bufs, acc_sh, sems):
    c = lax.axis_index("c")
    s = lax.axis_index("s")
    r0 = s * RB
    iota16 = lax.iota(jnp.int32, 16)
    for q in range(2):
        for p in range(2):
            cg = c * 2 + p
            pltpu.sync_copy(zeros16.at[pl.ds(r0, RB)],
                            acc_sh.at[pl.ds(r0, RB)])
            plsc.subcore_barrier()
            for half in range(2):
                for quart in range(2):
                    sl = s * 2 + half
                    c0 = quart * QCH
                    pltpu.sync_copy(src3.at[sl, pl.ds(c0, QCH)], idxs_v)
                    pltpu.sync_copy(dst3.at[sl, pl.ds(c0, QCH)], idxd_v)
                    pltpu.sync_copy(ex4.at[q, sl, pl.ds(c0, QCH)], ex_v)

                    def gat(j, b):
                        return pltpu.make_async_copy(
                            tabw.at[q, cg].at[idxs_v.at[j]], bufs.at[b],
                            sems.at[b])

                    for b in range(2):
                        gat(b, b).start()

                    def grp(g, carry):
                        for b in range(2):
                            j = g * 2 + b
                            gat(j, b).wait()

                            def rowgrp(v, carry2):
                                scale = ex_v[j, pl.ds(v * 16, 16)]
                                rows = (jnp.full((16,), v * 16, jnp.int32)
                                        + iota16)
                                for k in range(16):
                                    col = jnp.full((16,), k, jnp.int32)
                                    vals = plsc.load_gather(
                                        bufs.at[b], [rows, col])
                                    plsc.store_scatter(
                                        bufs.at[b], [rows, col],
                                        vals * scale)
                                return carry2

                            lax.fori_loop(0, 8, rowgrp, 0)
                            pltpu.sync_copy(bufs.at[b],
                                            acc_sh.at[idxd_v.at[j]],
                                            add=True)
                            nxt = j + 2

                            @pl.when(nxt < QCH)
                            def _():
                                gat(nxt, b).start()
                        return carry

                    lax.fori_loop(0, QCH // 2, grp, 0)
            plsc.subcore_barrier()
            pltpu.sync_copy(acc_sh.at[pl.ds(r0, RB)],
                            out.at[q, cg, pl.ds(r0, RB)])
            plsc.subcore_barrier()


def _full(x):
    return pl.BlockSpec(x.shape, lambda i: (0,) * x.ndim)


def _t_edge(ea, We1, be1, ge1, bte1, We2, be2, ge2, bte2, ae1, ae2):
    def body(ea_r, We1_r, be1_r, ge1_r, bte1_r, We2_r, be2_r, ge2_r, bte2_r,
             ae1_r, ae2_r, le_r):
        x = ea_r[...]
        e1 = jnp.maximum((_mm(x, We1_r[...]) + be1_r[...]) * ge1_r[...]
                         + bte1_r[...], 0.0)
        e2 = jnp.maximum((_mm(e1, We2_r[...]) + be2_r[...]) * ge2_r[...]
                         + bte2_r[...], 0.0)
        le_r[...] = jnp.concatenate(
            [_mm(e2, ae1_r[...].T), _mm(e2, ae2_r[...].T)], axis=1)

    args = (ea, We1, be1, ge1, bte1, We2, be2, ge2, bte2, ae1, ae2)
    return pl.pallas_call(
        body,
        grid=(EG,),
        in_specs=[pl.BlockSpec((EB, NE), lambda i: (i, 0))]
        + [_full(a) for a in args[1:]],
        out_specs=pl.BlockSpec((EB, 4), lambda i: (i, 0)),
        out_shape=jax.ShapeDtypeStruct((EP, 4), jnp.float32),
    )(*args)


def _t_front(dxp, cxp, degp, Wc, bc, Wg0, bg0, Wgc1):
    def body(dx_r, cx_r, deg_r, Wc_r, bc_r, Wg0_r, bg0_r, Wgc1_r,
             xgdc_r, xs1_r):
        x_d = dx_r[...][:, 6:20]
        cx = cx_r[...]
        Wcv = Wc_r[...]
        bcv = bc_r[...]
        parts = [jnp.maximum(_mm(cx[:, 10 * g:10 * g + 10], Wcv) + bcv, 0.0)
                 for g in range(3)]
        xgdc = jnp.concatenate([x_d] + parts, axis=1)
        xgdc_r[...] = xgdc
        xg = jnp.maximum(_mm(xgdc, Wg0_r[...]) + bg0_r[...], 0.0)
        deg = deg_r[0, :, 0] + deg_r[1, :, 0] + 1.0
        dinv = lax.rsqrt(deg)[:, None]
        xs = _mm(xg, Wgc1_r[...]) * dinv
        for g in range(4):
            xs1_r[g] = xs[:, 16 * g:16 * g + 16]

    args = (dxp, cxp, degp, Wc, bc, Wg0, bg0, Wgc1)
    return pl.pallas_call(
        body,
        grid=(NG,),
        in_specs=[pl.BlockSpec((RB, 20), lambda i: (i, 0)),
                  pl.BlockSpec((RB, 30), lambda i: (i, 0)),
                  pl.BlockSpec((2, RB, 8), lambda i: (0, i, 0))]
        + [_full(a) for a in args[3:]],
        out_specs=[pl.BlockSpec((RB, 26), lambda i: (i, 0)),
                   pl.BlockSpec((4, RB, 16), lambda i: (0, i, 0))],
        out_shape=[jax.ShapeDtypeStruct((NP, 26), jnp.float32),
                   jax.ShapeDtypeStruct((4, NP, 16), jnp.float32)],
    )(*args)


def _t_gcn_mid(acc1, xs1, degp, bgc1, Wgc2):
    def body(acc_r, xs_r, deg_r, bgc1_r, Wgc2_r, xg0_r, xs2_r):
        deg = deg_r[0, :, 0] + deg_r[1, :, 0] + 1.0
        dinv = lax.rsqrt(deg)[:, None]
        tot = jnp.concatenate([acc_r[g] + xs_r[g] for g in range(4)],
                              axis=1)
        xg0 = jnp.maximum(tot * dinv + bgc1_r[...], 0.0)
        xg0_r[...] = xg0
        xs = _mm(xg0, Wgc2_r[...]) * dinv
        for g in range(4):
            xs2_r[g] = xs[:, 16 * g:16 * g + 16]

    args = (acc1, xs1, degp, bgc1, Wgc2)
    return pl.pallas_call(
        body,
        grid=(NG,),
        in_specs=[pl.BlockSpec((4, RB, 16), lambda i: (0, i, 0)),
                  pl.BlockSpec((4, RB, 16), lambda i: (0, i, 0)),
                  pl.BlockSpec((2, RB, 8), lambda i: (0, i, 0)),
                  _full(bgc1), _full(Wgc2)],
        out_specs=[pl.BlockSpec((RB, 64), lambda i: (i, 0)),
                   pl.BlockSpec((4, RB, 16), lambda i: (0, i, 0))],
        out_shape=[jax.ShapeDtypeStruct((NP, 64), jnp.float32),
                   jax.ShapeDtypeStruct((4, NP, 16), jnp.float32)],
    )(*args)


def _t_main(acc2, xs2, degp, xg0, xgdc, bgc2, Wr1, br1, gr1, btr1,
            Wr2, br2, gr2, btr2, wcr, bcr, Wf, bf, Ws0, bs0, Wa1, as1, ad1):
    def body(acc_r, xs_r, deg_r, xg0_r, xgdc_r, bgc2_r, Wr1_r, br1_r, gr1_r,
             btr1_r, Wr2_r, br2_r, gr2_r, btr2_r, wcr_r, bcr_r, Wf_r, bf_r,
             Ws0_r, bs0_r, Wa1_r, as1_r, ad1_r, hci_r, xw1_r, tab1_r):
        deg = deg_r[0, :, 0] + deg_r[1, :, 0] + 1.0
        dinv = lax.rsqrt(deg)[:, None]
        tot = jnp.concatenate([acc_r[g] + xs_r[g] for g in range(4)],
                              axis=1)
        xg1 = jnp.maximum(tot * dinv + bgc2_r[...], 0.0)
        x = jnp.concatenate([xgdc_r[...], xg0_r[...] + xg1], axis=1)
        hd = x
        hd = jnp.maximum((_mm(hd, Wr1_r[...]) + br1_r[...]) * gr1_r[...]
                         + btr1_r[...], 0.0) + hd
        hd = jnp.maximum((_mm(hd, Wr2_r[...]) + br2_r[...]) * gr2_r[...]
                         + btr2_r[...], 0.0) + hd
        wcr_v = wcr_r[...]
        bcr_v = bcr_r[...]
        xl = x
        for i in range(2):
            sv = _mm(xl, wcr_v[i][:, None])
            xl = x * sv + bcr_v[i] + xl
        hci_r[...] = jnp.maximum(_mm(hd + xl, Wf_r[...]) + bf_r[...], 0.0)
        xsi = jnp.maximum(_mm(x, Ws0_r[...]) + bs0_r[...], 0.0)
        Wa = Wa1_r[...]
        asv = as1_r[...]
        adv = ad1_r[...]
        cols = []
        for h in range(2):
            xw = _mm(xsi, Wa[h])
            for g in range(4):
                xw1_r[h, g] = xw[:, 16 * g:16 * g + 16]
            cols.append(_mm(xw, asv[h][:, None]))
        for h in range(2):
            xw = _mm(xsi, Wa[h])
            cols.append(_mm(xw, adv[h][:, None]))
        tab1_r[...] = jnp.concatenate(
            cols + [jnp.zeros((cols[0].shape[0], 12), jnp.float32)], axis=1)

    args = (acc2, xs2, degp, xg0, xgdc, bgc2, Wr1, br1, gr1, btr1,
            Wr2, br2, gr2, btr2, wcr, bcr, Wf, bf, Ws0, bs0, Wa1, as1, ad1)
    RBM = RB // 2
    return pl.pallas_call(
        body,
        grid=(NP // RBM,),
        in_specs=[pl.BlockSpec((4, RBM, 16), lambda i: (0, i, 0)),
                  pl.BlockSpec((4, RBM, 16), lambda i: (0, i, 0)),
                  pl.BlockSpec((2, RBM, 8), lambda i: (0, i, 0)),
                  pl.BlockSpec((RBM, 64), lambda i: (i, 0)),
                  pl.BlockSpec((RBM, 26), lambda i: (i, 0))]
        + [_full(a) for a in args[5:]],
        out_specs=[pl.BlockSpec((RBM, 64), lambda i: (i, 0)),
                   pl.BlockSpec((2, 4, RBM, 16), lambda i: (0, 0, i, 0)),
                   pl.BlockSpec((RBM, 16), lambda i: (i, 0))],
        out_shape=[jax.ShapeDtypeStruct((N, 64), jnp.float32),
                   jax.ShapeDtypeStruct((2, 4, NP, 16), jnp.float32),
                   jax.ShapeDtypeStruct((NP, 16), jnp.float32)],
    )(*args)


def _t_logit(gath, le, le_off):
    def body(gs_r, gd_r, le_r, logit_r, bmax_r):
        sl = gs_r[0][:, 0:2]
        dl = gd_r[0][:, 2:4]
        lev = le_r[...][:, le_off:le_off + 2]
        z = sl + dl + lev
        z = jnp.where(z >= 0.0, z, 0.2 * z)
        i = pl.program_id(0)
        rid = i * EB + lax.broadcasted_iota(jnp.int32, (EB, 1), 0)
        z = jnp.where(rid < E, z, -1e30)
        logit_r[...] = z
        bmax_r[...] = jnp.full((1, 1, 8), jnp.max(z), jnp.float32)

    return pl.pallas_call(
        body,
        grid=(EG,),
        in_specs=[pl.BlockSpec((1, EB, 16), lambda i: (0, i, 0)),
                  pl.BlockSpec((1, EB, 16), lambda i: (1, i, 0)),
                  pl.BlockSpec((EB, 4), lambda i: (i, 0))],
        out_specs=[pl.BlockSpec((EB, 2), lambda i: (i, 0)),
                   pl.BlockSpec((1, 1, 8), lambda i: (i, 0, 0))],
        out_shape=[jax.ShapeDtypeStruct((EP, 2), jnp.float32),
                   jax.ShapeDtypeStruct((EG, 1, 8), jnp.float32)],
    )(gath, gath, le)


def _t_exp(logit, gmax):
    def body(logit_r, gmax_r, ex8_r):
        ex = jnp.exp(logit_r[...] - gmax_r[0, 0])
        ex8_r[...] = jnp.concatenate(
            [ex, jnp.zeros((EB, 6), jnp.float32)], axis=1)

    return pl.pallas_call(
        body,
        grid=(EG,),
        in_specs=[pl.BlockSpec((EB, 2), lambda i: (i, 0)),
                  _full(gmax)],
        out_specs=pl.BlockSpec((EB, 8), lambda i: (i, 0)),
        out_shape=jax.ShapeDtypeStruct((EP, 8), jnp.float32),
    )(logit, gmax)


def _t_combine(num, denp, bA, Wa2=None, as2=None, ad2=None, x_prev=None,
               make_tables=False, out_n=None):
    def body(*refs):
        if make_tables:
            (num_r, den_r, bA_r, Wa2_r, as2_r, ad2_r,
             xsi_r, xw2_r, tab2_r) = refs
        else:
            (num_r, den_r, bA_r, xp_r, hsi_r) = refs
        d0 = (den_r[0, :, 0] + den_r[1, :, 0] + 1e-16)[:, None]
        d1 = (den_r[0, :, 1] + den_r[1, :, 1] + 1e-16)[:, None]
        cols = []
        for cg in range(4):
            cols.append(0.5 * (num_r[0, cg] / d0 + num_r[1, cg] / d1))
        x = jnp.maximum(jnp.concatenate(cols, axis=1) + bA_r[...], 0.0)
        if make_tables:
            xsi_r[...] = x
            Wa = Wa2_r[...]
            asv = as2_r[...]
            adv = ad2_r[...]
            tcols = []
            for h in range(2):
                xw = _mm(x, Wa[h])
                for g in range(4):
                    xw2_r[h, g] = xw[:, 16 * g:16 * g + 16]
                tcols.append(_mm(xw, asv[h][:, None]))
            for h in range(2):
                xw = _mm(x, Wa[h])
                tcols.append(_mm(xw, adv[h][:, None]))
            tab2_r[...] = jnp.concatenate(
                tcols + [jnp.zeros((tcols[0].shape[0], 12), jnp.float32)],
                axis=1)
        else:
            hsi_r[...] = x + xp_r[...]

    RBC = RB // 4
    base_specs = [pl.BlockSpec((2, 4, RBC, 16), lambda i: (0, 0, i, 0)),
                  pl.BlockSpec((2, RBC, 8), lambda i: (0, i, 0)),
                  _full(bA)]
    if make_tables:
        args = (num, denp, bA, Wa2, as2, ad2)
        return pl.pallas_call(
            body,
            grid=(NP // RBC,),
            in_specs=base_specs + [_full(Wa2), _full(as2), _full(ad2)],
            out_specs=[pl.BlockSpec((RBC, 64), lambda i: (i, 0)),
                       pl.BlockSpec((2, 4, RBC, 16), lambda i: (0, 0, i, 0)),
                       pl.BlockSpec((RBC, 16), lambda i: (i, 0))],
            out_shape=[jax.ShapeDtypeStruct((NP, 64), jnp.float32),
                       jax.ShapeDtypeStruct((2, 4, NP, 16), jnp.float32),
                       jax.ShapeDtypeStruct((NP, 16), jnp.float32)],
        )(*args)
    args = (num, denp, bA, x_prev)
    return pl.pallas_call(
        body,
        grid=(NP // RBC,),
        in_specs=base_specs + [pl.BlockSpec((RBC, 64), lambda i: (i, 0))],
        out_specs=pl.BlockSpec((RBC, 64), lambda i: (i, 0)),
        out_shape=jax.ShapeDtypeStruct((out_n, 64), jnp.float32),
    )(*args)


def _t_head(h_ci, h_si, t, Wy0, by0, Wy1, by1, Wp0a, bp0a, Wp0b, bp0b,
            Wp1a, bp1a, Wp1b, bp1b, WpT, bpT):
    def body(hc_r, hs_r, t_r, Wy0_r, by0_r, Wy1_r, by1_r, Wp0a_r, bp0a_r,
             Wp0b_r, bp0b_r, Wp1a_r, bp1a_r, Wp1b_r, bp1b_r, WpT_r, bpT_r,
             py_r, pycf_r, py0_r, py1_r, pT_r):
        hc = hc_r[...]
        hs = hs_r[...]
        h = jnp.concatenate([hc, hs], axis=1)

        def smax(z):
            z = z - jnp.max(z, axis=1, keepdims=True)
            ez = jnp.exp(z)
            return ez / jnp.sum(ez, axis=1, keepdims=True)

        a0 = smax(_mm(h, Wy0_r[...]) + by0_r[...])
        a1 = smax(_mm(h, Wy1_r[...]) + by1_r[...])
        py0 = a0[:, :64] * hc + a0[:, 64:] * hs
        py1 = a1[:, :64] * hc + a1[:, 64:] * hs
        py0 = jax.nn.sigmoid(
            _mm(jnp.maximum(_mm(py0, Wp0a_r[...]) + bp0a_r[...], 0.0),
                Wp0b_r[...]) + bp0b_r[...])
        py1 = jax.nn.sigmoid(
            _mm(jnp.maximum(_mm(py1, Wp1a_r[...]) + bp1a_r[...], 0.0),
                Wp1b_r[...]) + bp1b_r[...])
        pT = jax.nn.sigmoid(_mm(hs, WpT_r[...]) + bpT_r[...])
        tv = t_r[...]
        py_r[...] = (1.0 - tv) * py0 + tv * py1
        pycf_r[...] = tv * py0 + (1.0 - tv) * py1
        py0_r[...] = py0
        py1_r[...] = py1
        pT_r[...] = pT

    args = (h_ci, h_si, t, Wy0, by0, Wy1, by1, Wp0a, bp0a, Wp0b, bp0b,
            Wp1a, bp1a, Wp1b, bp1b, WpT, bpT)
    HB = 2000
    o = pl.BlockSpec((HB, 1), lambda i: (i, 0))
    sd = jax.ShapeDtypeStruct((B, 1), jnp.float32)
    return pl.pallas_call(
        body,
        grid=(B // HB,),
        in_specs=[pl.BlockSpec((HB, 64), lambda i: (i, 0)),
                  pl.BlockSpec((HB, 64), lambda i: (i, 0)),
                  pl.BlockSpec((HB, 1), lambda i: (i, 0))]
        + [_full(a) for a in args[3:]],
        out_specs=[o, o, o, o, o],
        out_shape=[sd, sd, sd, sd, sd],
    )(*args)


def kernel(discrete_x, continous_x, edge_index, edge_attr, churn_date, t,
           Wc, bc, We1, be1, ge1, bte1, We2, be2, ge2, bte2,
           Wg0, bg0, Wgc1, bgc1, Wgc2, bgc2,
           Wr1, br1, gr1, btr1, Wr2, br2, gr2, btr2,
           wcr, bcr, Wf, bf, Ws0, bs0,
           Wa1, as1, ad1, ae1, bA1, Wa2, as2, ad2, ae2, bA2,
           Wy0, by0, Wy1, by1,
           Wp0a, bp0a, Wp0b, bp0b, Wp1a, bp1a, Wp1b, bp1b, WpT, bpT):
    f32 = jnp.float32
    src = edge_index[0].astype(jnp.int32)
    dst = edge_index[1].astype(jnp.int32)
    src_p = jnp.concatenate([src, jnp.zeros((EP - E,), jnp.int32)])
    dst_p = jnp.concatenate([dst, jnp.full((EP - E,), TRASH, jnp.int32)])
    src3 = src_p.reshape(32, CH, 128)
    dst3 = dst_p.reshape(32, CH, 128)
    idx_comb = jnp.stack([src3, dst3])
    zeros8 = jnp.zeros((NP, 8), f32)
    zeros16 = jnp.zeros((NP, 16), f32)
    ones8 = jnp.ones((128, 8), f32)
    dxp = jnp.pad(discrete_x, ((0, NP - N), (0, 0)))
    cxp = jnp.pad(continous_x, ((0, NP - N), (0, 0)))
    eap = jnp.pad(edge_attr, ((0, EP - E), (0, 0)))

    degp = _sc_degree(dst3, zeros8, ones8)
    le = _t_edge(eap, We1, be1, ge1, bte1, We2, be2, ge2, bte2, ae1, ae2)
    xgdc, xs1 = _t_front(dxp, cxp, degp, Wc, bc, Wg0, bg0, Wgc1)

    acc1 = _sc_gcn(xs1, src3, dst3, zeros16)
    xg0, xs2 = _t_gcn_mid(acc1, xs1, degp, bgc1, Wgc2)
    acc2 = _sc_gcn(xs2, src3, dst3, zeros16)

    h_ci, xw1, tab1 = _t_main(
        acc2, xs2, degp, xg0, xgdc, bgc2, Wr1, br1, gr1, btr1,
        Wr2, br2, gr2, btr2, wcr, bcr, Wf, bf, Ws0, bs0, Wa1, as1, ad1)

    g1 = _sc_dualgather(tab1, idx_comb)
    logit1, bmax1 = _t_logit(g1, le, 0)
    gmax1 = jnp.max(bmax1).reshape(1, 1)
    ex8_1 = _t_exp(logit1, gmax1)
    den1 = _sc_den(dst3, ex8_1, zeros8)
    ex4_1 = jnp.stack([ex8_1[:, 0].reshape(32, CH, 128),
                       ex8_1[:, 1].reshape(32, CH, 128)])
    num1 = _sc_egat(xw1, src3, dst3, ex4_1, zeros16)
    x_si0, xw2, tab2 = _t_combine(num1, den1, bA1, Wa2=Wa2, as2=as2,
                                  ad2=ad2, make_tables=True)

    g2 = _sc_dualgather(tab2, idx_comb)
    logit2, bmax2 = _t_logit(g2, le, 2)
    gmax2 = jnp.max(bmax2).reshape(1, 1)
    ex8_2 = _t_exp(logit2, gmax2)
    den2 = _sc_den(dst3, ex8_2, zeros8)
    ex4_2 = jnp.stack([ex8_2[:, 0].reshape(32, CH, 128),
                       ex8_2[:, 1].reshape(32, CH, 128)])
    num2 = _sc_egat(xw2, src3, dst3, ex4_2, zeros16)
    h_si = _t_combine(num2, den2, bA2, x_prev=x_si0, out_n=N)

    pred_y, pred_y_cf, py0, py1, pred_T = _t_head(
        h_ci, h_si, t, Wy0, by0, Wy1, by1, Wp0a, bp0a, Wp0b, bp0b,
        Wp1a, bp1a, Wp1b, bp1b, WpT, bpT)
    return (pred_y, pred_y_cf, py0, py1, pred_T, h_ci, h_si)


# speedup vs baseline: 24.7253x; 1.1549x over previous
import functools

import jax
import jax.numpy as jnp
from jax import lax
from jax.experimental import pallas as pl
from jax.experimental.pallas import tpu as pltpu
from jax.experimental.pallas import tpu_sc as plsc

N = 50000
E = 800000
B = 10000
H = 64
NE = 16
NH1 = 90

NP = 50176
RB = 3136
NG = NP // RB
TRASH = N

EP = 802816
SLICE = 25088
CH = 196
EB = 8192
EG = EP // EB

def _sc_kernel(**kw):
    def deco(fn):
        cache = {}

        def call(*args):
            if 'k' not in cache:
                mesh = plsc.VectorSubcoreMesh(
                    core_axis_name="c", subcore_axis_name="s",
                    num_cores=2, num_subcores=16)
                cache['k'] = pl.kernel(
                    fn, mesh=mesh,
                    compiler_params=pltpu.CompilerParams(
                        use_tc_tiling_on_sc=False,
                        needs_layout_passes=False),
                    **kw)
            return cache['k'](*args)

        return call

    return deco


def _mm(a, b):
    return jnp.dot(a, b, preferred_element_type=jnp.float32)


@_sc_kernel(
    out_type=jax.ShapeDtypeStruct((2, NP, 8), jnp.float32),
    scratch_types=[
        pltpu.VMEM((CH, 128), jnp.int32),
        pltpu.VMEM((128, 8), jnp.float32),
        pltpu.VMEM_SHARED((NP, 8), jnp.float32),
        pltpu.SemaphoreType.DMA,
    ],
)
def _sc_degree(dst3, zeros8, ones8, out, idx_v, ones_v, acc_sh, sem):
    c = lax.axis_index("c")
    s = lax.axis_index("s")
    wid = s * 2 + c
    r0 = s * RB
    pltpu.sync_copy(zeros8.at[pl.ds(r0, RB)], acc_sh.at[pl.ds(r0, RB)])
    pltpu.sync_copy(ones8, ones_v)
    pltpu.sync_copy(dst3.at[wid], idx_v)
    plsc.subcore_barrier()

    def sc(j):
        return pltpu.make_async_copy(ones_v, acc_sh.at[idx_v.at[j]], sem)

    def start(j, carry):
        sc(j).start(add=True)
        return carry

    def drain(j, carry):
        sc(j).wait()
        return carry

    lax.fori_loop(0, CH, start, 0)
    lax.fori_loop(0, CH, drain, 0)
    plsc.subcore_barrier()
    pltpu.sync_copy(acc_sh.at[pl.ds(r0, RB)], out.at[c, pl.ds(r0, RB)])


def _ring(nch, gat, scat, process=None, nbuf=7, prime=5):
    for b in range(prime):
        gat(b, b).start()

    def grp(g, carry):
        for i in range(nbuf):
            j = g * nbuf + i
            gat(j, i).wait()
            if process is not None:
                process(j, i)
            scat(j, i).start(add=scat.add)
            b2 = (i + prime) % nbuf

            @pl.when((j >= 2) & (j + prime < nch))
            def _():
                scat(j - 2, b2).wait()

            @pl.when(j + prime < nch)
            def _():
                gat(j + prime, b2).start()
        return carry

    lax.fori_loop(0, nch // nbuf, grp, 0)
    for b in range(nbuf):
        scat(nch - nbuf + b, b).wait()


@_sc_kernel(
    out_type=jax.ShapeDtypeStruct((2, NP, 8), jnp.float32),
    scratch_types=[
        pltpu.VMEM((CH, 128), jnp.int32),
        pltpu.VMEM((7, 128, 8), jnp.float32),
        pltpu.VMEM_SHARED((NP, 8), jnp.float32),
        pltpu.SemaphoreType.DMA((7,)),
        pltpu.SemaphoreType.DMA((7,)),
    ],
)
def _sc_den(dst3, ex8, zeros8, out, idx_v, bufs, acc_sh, gsems, ssems):
    c = lax.axis_index("c")
    s = lax.axis_index("s")
    wid = s * 2 + c
    base = wid * SLICE
    r0 = s * RB
    pltpu.sync_copy(zeros8.at[pl.ds(r0, RB)], acc_sh.at[pl.ds(r0, RB)])
    pltpu.sync_copy(dst3.at[wid], idx_v)
    plsc.subcore_barrier()

    def gat(j, b):
        return pltpu.make_async_copy(
            ex8.at[pl.ds(base + j * 128, 128)], bufs.at[b], gsems.at[b])

    def scat(j, b):
        return pltpu.make_async_copy(
            bufs.at[b], acc_sh.at[idx_v.at[j]], ssems.at[b])

    scat.add = True
    _ring(CH, gat, scat)
    plsc.subcore_barrier()
    pltpu.sync_copy(acc_sh.at[pl.ds(r0, RB)], out.at[c, pl.ds(r0, RB)])


@_sc_kernel(
    out_type=jax.ShapeDtypeStruct((4, NP, 16), jnp.float32),
    scratch_types=[
        pltpu.VMEM((CH, 128), jnp.int32),
        pltpu.VMEM((CH, 128), jnp.int32),
        pltpu.VMEM((7, 128, 16), jnp.float32),
        pltpu.VMEM_SHARED((NP, 16), jnp.float32),
        pltpu.SemaphoreType.DMA((7,)),
        pltpu.SemaphoreType.DMA((7,)),
    ],
)
def _sc_gcn(tab4, src3, dst3, zeros16, out, idxs_v, idxd_v, bufs, acc_sh,
            gsems, ssems):
    c = lax.axis_index("c")
    s = lax.axis_index("s")
    r0 = s * RB
    for p in range(2):
        cg = c * 2 + p
        pltpu.sync_copy(zeros16.at[pl.ds(r0, RB)], acc_sh.at[pl.ds(r0, RB)])
        plsc.subcore_barrier()
        for half in range(2):
            sl = s * 2 + half
            pltpu.sync_copy(src3.at[sl], idxs_v)
            pltpu.sync_copy(dst3.at[sl], idxd_v)

            def gat(j, b):
                return pltpu.make_async_copy(
                    tab4.at[cg].at[idxs_v.at[j]], bufs.at[b], gsems.at[b])

            def scat(j, b):
                return pltpu.make_async_copy(
                    bufs.at[b], acc_sh.at[idxd_v.at[j]], ssems.at[b])

            scat.add = True
            _ring(CH, gat, scat)
        plsc.subcore_barrier()
        pltpu.sync_copy(acc_sh.at[pl.ds(r0, RB)], out.at[cg, pl.ds(r0, RB)])
        plsc.subcore_barrier()


@_sc_kernel(
    out_type=jax.ShapeDtypeStruct((2, EP, 16), jnp.float32),
    scratch_types=[
        pltpu.VMEM((CH, 128), jnp.int32),
        pltpu.VMEM((7, 128, 16), jnp.float32),
        pltpu.SemaphoreType.DMA((7,)),
        pltpu.SemaphoreType.DMA((7,)),
    ],
)
def _sc_dualgather(tab, idx_comb, out, idx_v, bufs, gsems, ssems):
    c = lax.axis_index("c")
    s = lax.axis_index("s")
    for half in range(2):
        sl = s * 2 + half
        base = sl * SLICE
        pltpu.sync_copy(idx_comb.at[c, sl], idx_v)

        def gat(j, b):
            return pltpu.make_async_copy(
                tab.at[idx_v.at[j]], bufs.at[b], gsems.at[b])

        def scat(j, b):
            return pltpu.make_async_copy(
                bufs.at[b], out.at[c, pl.ds(base + j * 128, 128)],
                ssems.at[b])

        scat.add = False
        _ring(CH, gat, scat)


QCH = CH // 2


@_sc_kernel(
    out_type=jax.ShapeDtypeStruct((2, 4, NP, 16), jnp.float32),
    scratch_types=[
        pltpu.VMEM((QCH, 128), jnp.int32),
        pltpu.VMEM((QCH, 128), jnp.int32),
        pltpu.VMEM((QCH, 128), jnp.float32),
        pltpu.VMEM((7, 128, 16), jnp.float32),
        pltpu.VMEM_SHARED((NP, 16), jnp.float32),
        pltpu.SemaphoreType.DMA((7,)),
        pltpu.SemaphoreType.DMA((7,)),
    ],
)
def _sc_egat(tabw, src3, dst3, ex4, zeros16, out,
             idxs_v, idxd_v, ex_v, bufs, acc_sh, gsems, ssems):
    c = lax.axis_index("c")
    s = lax.axis_index("s")
    r0 = s * RB
    iota16 = lax.iota(jnp.int32, 16)

    def one_pass(qp, carry0):
        q = qp // 2
        cg = c * 2 + qp % 2
        pltpu.sync_copy(zeros16.at[pl.ds(r0, RB)], acc_sh.at[pl.ds(r0, RB)])
        plsc.subcore_barrier()

        def one_seg(seg, carry1):
            sl = s * 2 + seg // 2
            c0 = (seg % 2) * QCH
            pltpu.sync_copy(src3.at[sl, pl.ds(c0, QCH)], idxs_v)
            pltpu.sync_copy(dst3.at[sl, pl.ds(c0, QCH)], idxd_v)
            pltpu.sync_copy(ex4.at[q, sl, pl.ds(c0, QCH)], ex_v)

            def gat(j, b):
                return pltpu.make_async_copy(
                    tabw.at[q, cg].at[idxs_v.at[j]], bufs.at[b],
                    gsems.at[b])

            def scat(j, b):
                return pltpu.make_async_copy(
                    bufs.at[b], acc_sh.at[idxd_v.at[j]], ssems.at[b])

            scat.add = True

            def process(j, b):
                jfull = jnp.full((16,), j, jnp.int32)
                buf = bufs.at[b]

                def rowgrp(v, carry2):
                    for l in range(16):
                        rfull = jnp.full((16,), v * 16 + l, jnp.int32)
                        scale = plsc.load_gather(ex_v, [jfull, rfull])
                        vals = plsc.load_gather(buf, [rfull, iota16])
                        plsc.store_scatter(buf, [rfull, iota16],
                                           vals * scale)
                    return carry2

                lax.fori_loop(0, 8, rowgrp, 0)

            _ring(QCH, gat, scat, process=process)
            return carry1

        lax.fori_loop(0, 4, one_seg, 0)
        plsc.subcore_barrier()
        pltpu.sync_copy(acc_sh.at[pl.ds(r0, RB)],
                        out.at[q, cg, pl.ds(r0, RB)])
        plsc.subcore_barrier()
        return carry0

    lax.fori_loop(0, 4, one_pass, 0)


def _full(x):
    return pl.BlockSpec(x.shape, lambda i: (0,) * x.ndim)


def _t_edge(ea, We1, be1, ge1, bte1, We2, be2, ge2, bte2, ae1, ae2):
    def body(ea_r, We1_r, be1_r, ge1_r, bte1_r, We2_r, be2_r, ge2_r, bte2_r,
             ae1_r, ae2_r, le_r):
        x = ea_r[...]
        e1 = jnp.maximum((_mm(x, We1_r[...]) + be1_r[...]) * ge1_r[...]
                         + bte1_r[...], 0.0)
        e2 = jnp.maximum((_mm(e1, We2_r[...]) + be2_r[...]) * ge2_r[...]
                         + bte2_r[...], 0.0)
        le_r[...] = jnp.concatenate(
            [_mm(e2, ae1_r[...].T), _mm(e2, ae2_r[...].T)], axis=1)

    args = (ea, We1, be1, ge1, bte1, We2, be2, ge2, bte2, ae1, ae2)
    return pl.pallas_call(
        body,
        grid=(EG,),
        in_specs=[pl.BlockSpec((EB, NE), lambda i: (i, 0))]
        + [_full(a) for a in args[1:]],
        out_specs=pl.BlockSpec((EB, 4), lambda i: (i, 0)),
        out_shape=jax.ShapeDtypeStruct((EP, 4), jnp.float32),
    )(*args)


def _t_front(dxp, cxp, degp, Wc, bc, Wg0, bg0, Wgc1):
    def body(dx_r, cx_r, deg_r, Wc_r, bc_r, Wg0_r, bg0_r, Wgc1_r,
             xgdc_r, xs1_r):
        x_d = dx_r[...][:, 6:20]
        cx = cx_r[...]
        Wcv = Wc_r[...]
        bcv = bc_r[...]
        parts = [jnp.maximum(_mm(cx[:, 10 * g:10 * g + 10], Wcv) + bcv, 0.0)
                 for g in range(3)]
        xgdc = jnp.concatenate([x_d] + parts, axis=1)
        xgdc_r[...] = xgdc
        xg = jnp.maximum(_mm(xgdc, Wg0_r[...]) + bg0_r[...], 0.0)
        deg = deg_r[0, :, 0] + deg_r[1, :, 0] + 1.0
        dinv = lax.rsqrt(deg)[:, None]
        xs = _mm(xg, Wgc1_r[...]) * dinv
        for g in range(4):
            xs1_r[g] = xs[:, 16 * g:16 * g + 16]

    args = (dxp, cxp, degp, Wc, bc, Wg0, bg0, Wgc1)
    return pl.pallas_call(
        body,
        grid=(NG,),
        in_specs=[pl.BlockSpec((RB, 20), lambda i: (i, 0)),
                  pl.BlockSpec((RB, 30), lambda i: (i, 0)),
                  pl.BlockSpec((2, RB, 8), lambda i: (0, i, 0))]
        + [_full(a) for a in args[3:]],
        out_specs=[pl.BlockSpec((RB, 26), lambda i: (i, 0)),
                   pl.BlockSpec((4, RB, 16), lambda i: (0, i, 0))],
        out_shape=[jax.ShapeDtypeStruct((NP, 26), jnp.float32),
                   jax.ShapeDtypeStruct((4, NP, 16), jnp.float32)],
    )(*args)


def _t_gcn_mid(acc1, xs1, degp, bgc1, Wgc2):
    def body(acc_r, xs_r, deg_r, bgc1_r, Wgc2_r, xg0_r, xs2_r):
        deg = deg_r[0, :, 0] + deg_r[1, :, 0] + 1.0
        dinv = lax.rsqrt(deg)[:, None]
        tot = jnp.concatenate([acc_r[g] + xs_r[g] for g in range(4)],
                              axis=1)
        xg0 = jnp.maximum(tot * dinv + bgc1_r[...], 0.0)
        xg0_r[...] = xg0
        xs = _mm(xg0, Wgc2_r[...]) * dinv
        for g in range(4):
            xs2_r[g] = xs[:, 16 * g:16 * g + 16]

    args = (acc1, xs1, degp, bgc1, Wgc2)
    return pl.pallas_call(
        body,
        grid=(NG,),
        in_specs=[pl.BlockSpec((4, RB, 16), lambda i: (0, i, 0)),
                  pl.BlockSpec((4, RB, 16), lambda i: (0, i, 0)),
                  pl.BlockSpec((2, RB, 8), lambda i: (0, i, 0)),
                  _full(bgc1), _full(Wgc2)],
        out_specs=[pl.BlockSpec((RB, 64), lambda i: (i, 0)),
                   pl.BlockSpec((4, RB, 16), lambda i: (0, i, 0))],
        out_shape=[jax.ShapeDtypeStruct((NP, 64), jnp.float32),
                   jax.ShapeDtypeStruct((4, NP, 16), jnp.float32)],
    )(*args)


def _t_main(acc2, xs2, degp, xg0, xgdc, bgc2, Wr1, br1, gr1, btr1,
            Wr2, br2, gr2, btr2, wcr, bcr, Wf, bf, Ws0, bs0, Wa1, as1, ad1):
    def body(acc_r, xs_r, deg_r, xg0_r, xgdc_r, bgc2_r, Wr1_r, br1_r, gr1_r,
             btr1_r, Wr2_r, br2_r, gr2_r, btr2_r, wcr_r, bcr_r, Wf_r, bf_r,
             Ws0_r, bs0_r, Wa1_r, as1_r, ad1_r, hci_r, xw1_r, tab1_r):
        deg = deg_r[0, :, 0] + deg_r[1, :, 0] + 1.0
        dinv = lax.rsqrt(deg)[:, None]
        tot = jnp.concatenate([acc_r[g] + xs_r[g] for g in range(4)],
                              axis=1)
        xg1 = jnp.maximum(tot * dinv + bgc2_r[...], 0.0)
        x = jnp.concatenate([xgdc_r[...], xg0_r[...] + xg1], axis=1)
        hd = x
        hd = jnp.maximum((_mm(hd, Wr1_r[...]) + br1_r[...]) * gr1_r[...]
                         + btr1_r[...], 0.0) + hd
        hd = jnp.maximum((_mm(hd, Wr2_r[...]) + br2_r[...]) * gr2_r[...]
                         + btr2_r[...], 0.0) + hd
        wcr_v = wcr_r[...]
        bcr_v = bcr_r[...]
        xl = x
        for i in range(2):
            sv = _mm(xl, wcr_v[i][:, None])
            xl = x * sv + bcr_v[i] + xl
        hci_r[...] = jnp.maximum(_mm(hd + xl, Wf_r[...]) + bf_r[...], 0.0)
        xsi = jnp.maximum(_mm(x, Ws0_r[...]) + bs0_r[...], 0.0)
        Wa = Wa1_r[...]
        asv = as1_r[...]
        adv = ad1_r[...]
        cols = []
        for h in range(2):
            xw = _mm(xsi, Wa[h])
            for g in range(4):
                xw1_r[h, g] = xw[:, 16 * g:16 * g + 16]
            cols.append(_mm(xw, asv[h][:, None]))
        for h in range(2):
            xw = _mm(xsi, Wa[h])
            cols.append(_mm(xw, adv[h][:, None]))
        tab1_r[...] = jnp.concatenate(
            cols + [jnp.zeros((cols[0].shape[0], 12), jnp.float32)], axis=1)

    args = (acc2, xs2, degp, xg0, xgdc, bgc2, Wr1, br1, gr1, btr1,
            Wr2, br2, gr2, btr2, wcr, bcr, Wf, bf, Ws0, bs0, Wa1, as1, ad1)
    RBM = RB // 2
    return pl.pallas_call(
        body,
        grid=(NP // RBM,),
        in_specs=[pl.BlockSpec((4, RBM, 16), lambda i: (0, i, 0)),
                  pl.BlockSpec((4, RBM, 16), lambda i: (0, i, 0)),
                  pl.BlockSpec((2, RBM, 8), lambda i: (0, i, 0)),
                  pl.BlockSpec((RBM, 64), lambda i: (i, 0)),
                  pl.BlockSpec((RBM, 26), lambda i: (i, 0))]
        + [_full(a) for a in args[5:]],
        out_specs=[pl.BlockSpec((RBM, 64), lambda i: (i, 0)),
                   pl.BlockSpec((2, 4, RBM, 16), lambda i: (0, 0, i, 0)),
                   pl.BlockSpec((RBM, 16), lambda i: (i, 0))],
        out_shape=[jax.ShapeDtypeStruct((N, 64), jnp.float32),
                   jax.ShapeDtypeStruct((2, 4, NP, 16), jnp.float32),
                   jax.ShapeDtypeStruct((NP, 16), jnp.float32)],
    )(*args)


def _t_logit(gath, le, le_off):
    def body(gs_r, gd_r, le_r, logit_r, bmax_r):
        sl = gs_r[0][:, 0:2]
        dl = gd_r[0][:, 2:4]
        lev = le_r[...][:, le_off:le_off + 2]
        z = sl + dl + lev
        z = jnp.where(z >= 0.0, z, 0.2 * z)
        i = pl.program_id(0)
        rid = i * EB + lax.broadcasted_iota(jnp.int32, (EB, 1), 0)
        z = jnp.where(rid < E, z, -1e30)
        logit_r[...] = z
        bmax_r[...] = jnp.full((1, 1, 8), jnp.max(z), jnp.float32)

    return pl.pallas_call(
        body,
        grid=(EG,),
        in_specs=[pl.BlockSpec((1, EB, 16), lambda i: (0, i, 0)),
                  pl.BlockSpec((1, EB, 16), lambda i: (1, i, 0)),
                  pl.BlockSpec((EB, 4), lambda i: (i, 0))],
        out_specs=[pl.BlockSpec((EB, 2), lambda i: (i, 0)),
                   pl.BlockSpec((1, 1, 8), lambda i: (i, 0, 0))],
        out_shape=[jax.ShapeDtypeStruct((EP, 2), jnp.float32),
                   jax.ShapeDtypeStruct((EG, 1, 8), jnp.float32)],
    )(gath, gath, le)


def _t_exp(logit, gmax):
    def body(logit_r, gmax_r, ex8_r):
        ex = jnp.exp(logit_r[...] - gmax_r[0, 0])
        ex8_r[...] = jnp.concatenate(
            [ex, jnp.zeros((EB, 6), jnp.float32)], axis=1)

    return pl.pallas_call(
        body,
        grid=(EG,),
        in_specs=[pl.BlockSpec((EB, 2), lambda i: (i, 0)),
                  _full(gmax)],
        out_specs=pl.BlockSpec((EB, 8), lambda i: (i, 0)),
        out_shape=jax.ShapeDtypeStruct((EP, 8), jnp.float32),
    )(logit, gmax)


def _t_combine(num, denp, bA, Wa2=None, as2=None, ad2=None, x_prev=None,
               make_tables=False, out_n=None):
    def body(*refs):
        if make_tables:
            (num_r, den_r, bA_r, Wa2_r, as2_r, ad2_r,
             xsi_r, xw2_r, tab2_r) = refs
        else:
            (num_r, den_r, bA_r, xp_r, hsi_r) = refs
        d0 = (den_r[0, :, 0] + den_r[1, :, 0] + 1e-16)[:, None]
        d1 = (den_r[0, :, 1] + den_r[1, :, 1] + 1e-16)[:, None]
        cols = []
        for cg in range(4):
            cols.append(0.5 * (num_r[0, cg] / d0 + num_r[1, cg] / d1))
        x = jnp.maximum(jnp.concatenate(cols, axis=1) + bA_r[...], 0.0)
        if make_tables:
            xsi_r[...] = x
            Wa = Wa2_r[...]
            asv = as2_r[...]
            adv = ad2_r[...]
            tcols = []
            for h in range(2):
                xw = _mm(x, Wa[h])
                for g in range(4):
                    xw2_r[h, g] = xw[:, 16 * g:16 * g + 16]
                tcols.append(_mm(xw, asv[h][:, None]))
            for h in range(2):
                xw = _mm(x, Wa[h])
                tcols.append(_mm(xw, adv[h][:, None]))
            tab2_r[...] = jnp.concatenate(
                tcols + [jnp.zeros((tcols[0].shape[0], 12), jnp.float32)],
                axis=1)
        else:
            hsi_r[...] = x + xp_r[...]

    RBC = RB // 4
    base_specs = [pl.BlockSpec((2, 4, RBC, 16), lambda i: (0, 0, i, 0)),
                  pl.BlockSpec((2, RBC, 8), lambda i: (0, i, 0)),
                  _full(bA)]
    if make_tables:
        args = (num, denp, bA, Wa2, as2, ad2)
        return pl.pallas_call(
            body,
            grid=(NP // RBC,),
            in_specs=base_specs + [_full(Wa2), _full(as2), _full(ad2)],
            out_specs=[pl.BlockSpec((RBC, 64), lambda i: (i, 0)),
                       pl.BlockSpec((2, 4, RBC, 16), lambda i: (0, 0, i, 0)),
                       pl.BlockSpec((RBC, 16), lambda i: (i, 0))],
            out_shape=[jax.ShapeDtypeStruct((NP, 64), jnp.float32),
                       jax.ShapeDtypeStruct((2, 4, NP, 16), jnp.float32),
                       jax.ShapeDtypeStruct((NP, 16), jnp.float32)],
        )(*args)
    args = (num, denp, bA, x_prev)
    return pl.pallas_call(
        body,
        grid=(NP // RBC,),
        in_specs=base_specs + [pl.BlockSpec((RBC, 64), lambda i: (i, 0))],
        out_specs=pl.BlockSpec((RBC, 64), lambda i: (i, 0)),
        out_shape=jax.ShapeDtypeStruct((out_n, 64), jnp.float32),
    )(*args)


def _t_head(h_ci, h_si, t, Wy0, by0, Wy1, by1, Wp0a, bp0a, Wp0b, bp0b,
            Wp1a, bp1a, Wp1b, bp1b, WpT, bpT):
    def body(hc_r, hs_r, t_r, Wy0_r, by0_r, Wy1_r, by1_r, Wp0a_r, bp0a_r,
             Wp0b_r, bp0b_r, Wp1a_r, bp1a_r, Wp1b_r, bp1b_r, WpT_r, bpT_r,
             py_r, pycf_r, py0_r, py1_r, pT_r):
        hc = hc_r[...]
        hs = hs_r[...]
        h = jnp.concatenate([hc, hs], axis=1)

        def smax(z):
            z = z - jnp.max(z, axis=1, keepdims=True)
            ez = jnp.exp(z)
            return ez / jnp.sum(ez, axis=1, keepdims=True)

        a0 = smax(_mm(h, Wy0_r[...]) + by0_r[...])
        a1 = smax(_mm(h, Wy1_r[...]) + by1_r[...])
        py0 = a0[:, :64] * hc + a0[:, 64:] * hs
        py1 = a1[:, :64] * hc + a1[:, 64:] * hs
        py0 = jax.nn.sigmoid(
            _mm(jnp.maximum(_mm(py0, Wp0a_r[...]) + bp0a_r[...], 0.0),
                Wp0b_r[...]) + bp0b_r[...])
        py1 = jax.nn.sigmoid(
            _mm(jnp.maximum(_mm(py1, Wp1a_r[...]) + bp1a_r[...], 0.0),
                Wp1b_r[...]) + bp1b_r[...])
        pT = jax.nn.sigmoid(_mm(hs, WpT_r[...]) + bpT_r[...])
        tv = t_r[...]
        py_r[...] = (1.0 - tv) * py0 + tv * py1
        pycf_r[...] = tv * py0 + (1.0 - tv) * py1
        py0_r[...] = py0
        py1_r[...] = py1
        pT_r[...] = pT

    args = (h_ci, h_si, t, Wy0, by0, Wy1, by1, Wp0a, bp0a, Wp0b, bp0b,
            Wp1a, bp1a, Wp1b, bp1b, WpT, bpT)
    HB = 2000
    o = pl.BlockSpec((HB, 1), lambda i: (i, 0))
    sd = jax.ShapeDtypeStruct((B, 1), jnp.float32)
    return pl.pallas_call(
        body,
        grid=(B // HB,),
        in_specs=[pl.BlockSpec((HB, 64), lambda i: (i, 0)),
                  pl.BlockSpec((HB, 64), lambda i: (i, 0)),
                  pl.BlockSpec((HB, 1), lambda i: (i, 0))]
        + [_full(a) for a in args[3:]],
        out_specs=[o, o, o, o, o],
        out_shape=[sd, sd, sd, sd, sd],
    )(*args)


def kernel(discrete_x, continous_x, edge_index, edge_attr, churn_date, t,
           Wc, bc, We1, be1, ge1, bte1, We2, be2, ge2, bte2,
           Wg0, bg0, Wgc1, bgc1, Wgc2, bgc2,
           Wr1, br1, gr1, btr1, Wr2, br2, gr2, btr2,
           wcr, bcr, Wf, bf, Ws0, bs0,
           Wa1, as1, ad1, ae1, bA1, Wa2, as2, ad2, ae2, bA2,
           Wy0, by0, Wy1, by1,
           Wp0a, bp0a, Wp0b, bp0b, Wp1a, bp1a, Wp1b, bp1b, WpT, bpT):
    f32 = jnp.float32
    src = edge_index[0].astype(jnp.int32)
    dst = edge_index[1].astype(jnp.int32)
    src_p = jnp.concatenate([src, jnp.zeros((EP - E,), jnp.int32)])
    dst_p = jnp.concatenate([dst, jnp.full((EP - E,), TRASH, jnp.int32)])
    src3 = src_p.reshape(32, CH, 128)
    dst3 = dst_p.reshape(32, CH, 128)
    idx_comb = jnp.stack([src3, dst3])
    zeros8 = jnp.zeros((NP, 8), f32)
    zeros16 = jnp.zeros((NP, 16), f32)
    ones8 = jnp.ones((128, 8), f32)
    dxp = jnp.pad(discrete_x, ((0, NP - N), (0, 0)))
    cxp = jnp.pad(continous_x, ((0, NP - N), (0, 0)))
    eap = jnp.pad(edge_attr, ((0, EP - E), (0, 0)))

    degp = _sc_degree(dst3, zeros8, ones8)
    le = _t_edge(eap, We1, be1, ge1, bte1, We2, be2, ge2, bte2, ae1, ae2)
    xgdc, xs1 = _t_front(dxp, cxp, degp, Wc, bc, Wg0, bg0, Wgc1)

    acc1 = _sc_gcn(xs1, src3, dst3, zeros16)
    xg0, xs2 = _t_gcn_mid(acc1, xs1, degp, bgc1, Wgc2)
    acc2 = _sc_gcn(xs2, src3, dst3, zeros16)

    h_ci, xw1, tab1 = _t_main(
        acc2, xs2, degp, xg0, xgdc, bgc2, Wr1, br1, gr1, btr1,
        Wr2, br2, gr2, btr2, wcr, bcr, Wf, bf, Ws0, bs0, Wa1, as1, ad1)

    g1 = _sc_dualgather(tab1, idx_comb)
    logit1, bmax1 = _t_logit(g1, le, 0)
    gmax1 = jnp.max(bmax1).reshape(1, 1)
    ex8_1 = _t_exp(logit1, gmax1)
    den1 = _sc_den(dst3, ex8_1, zeros8)
    ex4_1 = jnp.stack([ex8_1[:, 0].reshape(32, CH, 128),
                       ex8_1[:, 1].reshape(32, CH, 128)])
    num1 = _sc_egat(xw1, src3, dst3, ex4_1, zeros16)
    x_si0, xw2, tab2 = _t_combine(num1, den1, bA1, Wa2=Wa2, as2=as2,
                                  ad2=ad2, make_tables=True)

    g2 = _sc_dualgather(tab2, idx_comb)
    logit2, bmax2 = _t_logit(g2, le, 2)
    gmax2 = jnp.max(bmax2).reshape(1, 1)
    ex8_2 = _t_exp(logit2, gmax2)
    den2 = _sc_den(dst3, ex8_2, zeros8)
    ex4_2 = jnp.stack([ex8_2[:, 0].reshape(32, CH, 128),
                       ex8_2[:, 1].reshape(32, CH, 128)])
    num2 = _sc_egat(xw2, src3, dst3, ex4_2, zeros16)
    h_si = _t_combine(num2, den2, bA2, x_prev=x_si0, out_n=N)

    pred_y, pred_y_cf, py0, py1, pred_T = _t_head(
        h_ci, h_si, t, Wy0, by0, Wy1, by1, Wp0a, bp0a, Wp0b, bp0b,
        Wp1a, bp1a, Wp1b, bp1b, WpT, bpT)
    return (pred_y, pred_y_cf, py0, py1, pred_T, h_ci, h_si)


# speedup vs baseline: 26.5282x; 1.0729x over previous
import functools

import jax
import jax.numpy as jnp
from jax import lax
from jax.experimental import pallas as pl
from jax.experimental.pallas import tpu as pltpu
from jax.experimental.pallas import tpu_sc as plsc

N = 50000
E = 800000
B = 10000
H = 64
NE = 16
NH1 = 90

NP = 50176
RB = 3136
NG = NP // RB
TRASH = N

EP = 802816
SLICE = 25088
CH = 196
EB = 8192
EG = EP // EB

def _sc_kernel(**kw):
    def deco(fn):
        cache = {}

        def call(*args):
            if 'k' not in cache:
                mesh = plsc.VectorSubcoreMesh(
                    core_axis_name="c", subcore_axis_name="s",
                    num_cores=2, num_subcores=16)
                cache['k'] = pl.kernel(
                    fn, mesh=mesh,
                    compiler_params=pltpu.CompilerParams(
                        use_tc_tiling_on_sc=False,
                        needs_layout_passes=False),
                    **kw)
            return cache['k'](*args)

        return call

    return deco


def _mm(a, b):
    return jnp.dot(a, b, preferred_element_type=jnp.float32)


@_sc_kernel(
    out_type=jax.ShapeDtypeStruct((2, NP, 8), jnp.float32),
    scratch_types=[
        pltpu.VMEM((CH, 128), jnp.int32),
        pltpu.VMEM((128, 8), jnp.float32),
        pltpu.VMEM_SHARED((NP, 8), jnp.float32),
        pltpu.SemaphoreType.DMA,
    ],
)
def _sc_degree(dst3, zeros8, ones8, out, idx_v, ones_v, acc_sh, sem):
    c = lax.axis_index("c")
    s = lax.axis_index("s")
    wid = s * 2 + c
    r0 = s * RB
    pltpu.sync_copy(zeros8.at[pl.ds(r0, RB)], acc_sh.at[pl.ds(r0, RB)])
    pltpu.sync_copy(ones8, ones_v)
    pltpu.sync_copy(dst3.at[wid], idx_v)
    plsc.subcore_barrier()

    def sc(j):
        return pltpu.make_async_copy(ones_v, acc_sh.at[idx_v.at[j]], sem)

    def start(j, carry):
        sc(j).start(add=True)
        return carry

    def drain(j, carry):
        sc(j).wait()
        return carry

    lax.fori_loop(0, CH, start, 0)
    lax.fori_loop(0, CH, drain, 0)
    plsc.subcore_barrier()
    pltpu.sync_copy(acc_sh.at[pl.ds(r0, RB)], out.at[c, pl.ds(r0, RB)])


def _ring(nch, gat, scat, process=None, nbuf=7, prime=5):
    def aslist(d):
        return d if isinstance(d, (list, tuple)) else [d]

    for b in range(prime):
        for d in aslist(gat(b, b)):
            d.start()

    def grp(g, carry):
        for i in range(nbuf):
            j = g * nbuf + i
            for d in aslist(gat(j, i)):
                d.wait()
            if process is not None:
                process(j, i)
            scat(j, i).start(add=scat.add)
            b2 = (i + prime) % nbuf

            @pl.when((j >= 2) & (j + prime < nch))
            def _():
                scat(j - 2, b2).wait()

            @pl.when(j + prime < nch)
            def _():
                for d in aslist(gat(j + prime, b2)):
                    d.start()
        return carry

    lax.fori_loop(0, nch // nbuf, grp, 0)
    for b in range(nbuf):
        scat(nch - nbuf + b, b).wait()


@_sc_kernel(
    out_type=jax.ShapeDtypeStruct((2, NP, 8), jnp.float32),
    scratch_types=[
        pltpu.VMEM((CH, 128), jnp.int32),
        pltpu.VMEM((7, 128, 8), jnp.float32),
        pltpu.VMEM((16,), jnp.float32),
        pltpu.VMEM_SHARED((NP, 8), jnp.float32),
        pltpu.SemaphoreType.DMA((7,)),
        pltpu.SemaphoreType.DMA((7,)),
    ],
)
def _sc_den(dst3, logit8, gm16, zeros8, out, idx_v, bufs, gm_v, acc_sh,
            gsems, ssems):
    c = lax.axis_index("c")
    s = lax.axis_index("s")
    wid = s * 2 + c
    base = wid * SLICE
    r0 = s * RB
    iota16 = lax.iota(jnp.int32, 16)
    pltpu.sync_copy(zeros8.at[pl.ds(r0, RB)], acc_sh.at[pl.ds(r0, RB)])
    pltpu.sync_copy(dst3.at[wid], idx_v)
    pltpu.sync_copy(gm16, gm_v)
    plsc.subcore_barrier()
    gm = gm_v[...]

    def gat(j, b):
        return pltpu.make_async_copy(
            logit8.at[pl.ds(base + j * 128, 128)], bufs.at[b], gsems.at[b])

    def scat(j, b):
        return pltpu.make_async_copy(
            bufs.at[b], acc_sh.at[idx_v.at[j]], ssems.at[b])

    scat.add = True

    def process(j, b):
        def rowgrp(v, carry2):
            rows = jnp.full((16,), v * 16, jnp.int32) + iota16
            for q in range(2):
                qf = jnp.full((16,), q, jnp.int32)
                z = plsc.load_gather(bufs.at[b], [rows, qf])
                plsc.store_scatter(bufs.at[b], [rows, qf], jnp.exp(z - gm))
            for k in range(2, 8):
                plsc.store_scatter(
                    bufs.at[b], [rows, jnp.full((16,), k, jnp.int32)],
                    jnp.zeros((16,), jnp.float32))
            return carry2

        lax.fori_loop(0, 8, rowgrp, 0)

    _ring(CH, gat, scat, process=process)
    plsc.subcore_barrier()
    pltpu.sync_copy(acc_sh.at[pl.ds(r0, RB)], out.at[c, pl.ds(r0, RB)])


@_sc_kernel(
    out_type=jax.ShapeDtypeStruct((4, NP, 16), jnp.float32),
    scratch_types=[
        pltpu.VMEM((CH, 128), jnp.int32),
        pltpu.VMEM((CH, 128), jnp.int32),
        pltpu.VMEM((7, 128, 16), jnp.float32),
        pltpu.VMEM_SHARED((NP, 16), jnp.float32),
        pltpu.SemaphoreType.DMA((7,)),
        pltpu.SemaphoreType.DMA((7,)),
    ],
)
def _sc_gcn(tab4, src3, dst3, zeros16, out, idxs_v, idxd_v, bufs, acc_sh,
            gsems, ssems):
    c = lax.axis_index("c")
    s = lax.axis_index("s")
    r0 = s * RB
    for p in range(2):
        cg = c * 2 + p
        pltpu.sync_copy(zeros16.at[pl.ds(r0, RB)], acc_sh.at[pl.ds(r0, RB)])
        plsc.subcore_barrier()
        for half in range(2):
            sl = s * 2 + half
            pltpu.sync_copy(src3.at[sl], idxs_v)
            pltpu.sync_copy(dst3.at[sl], idxd_v)

            def gat(j, b):
                return pltpu.make_async_copy(
                    tab4.at[cg].at[idxs_v.at[j]], bufs.at[b], gsems.at[b])

            def scat(j, b):
                return pltpu.make_async_copy(
                    bufs.at[b], acc_sh.at[idxd_v.at[j]], ssems.at[b])

            scat.add = True
            _ring(CH, gat, scat)
        plsc.subcore_barrier()
        pltpu.sync_copy(acc_sh.at[pl.ds(r0, RB)], out.at[cg, pl.ds(r0, RB)])
        plsc.subcore_barrier()


@_sc_kernel(
    out_type=[jax.ShapeDtypeStruct((EP, 8), jnp.float32),
              jax.ShapeDtypeStruct((2, 16, 16), jnp.float32)],
    scratch_types=[
        pltpu.VMEM((CH, 128), jnp.int32),
        pltpu.VMEM((CH, 128), jnp.int32),
        pltpu.VMEM((7, 128, 16), jnp.float32),
        pltpu.VMEM((7, 128, 16), jnp.float32),
        pltpu.VMEM((7, 128, 8), jnp.float32),
        pltpu.VMEM((7, 128, 8), jnp.float32),
        pltpu.VMEM((16,), jnp.float32),
        pltpu.VMEM((16,), jnp.int32),
        pltpu.SemaphoreType.DMA((7,)),
        pltpu.SemaphoreType.DMA((7,)),
        pltpu.SemaphoreType.DMA((7,)),
        pltpu.SemaphoreType.DMA((7,)),
    ],
)
def _sc_logit(tabs, src3, dst3, le8, loff16, out, maxout,
              idxs_v, idxd_v, bs, bd, bl, bo, mx_v, lo_v,
              sems, semd, seml, semo):
    c = lax.axis_index("c")
    s = lax.axis_index("s")
    wid = s * 2 + c
    base = wid * SLICE
    iota16 = lax.iota(jnp.int32, 16)
    pltpu.sync_copy(src3.at[wid], idxs_v)
    pltpu.sync_copy(dst3.at[wid], idxd_v)
    pltpu.sync_copy(loff16, lo_v)
    mx_v[...] = jnp.full((16,), -1e30, jnp.float32)
    neg = jnp.full((16,), -1e30, jnp.float32)
    for b in range(7):
        def pre(v, carry):
            rows = jnp.full((16,), v * 16, jnp.int32) + iota16
            for k in range(2, 8):
                plsc.store_scatter(
                    bo.at[b], [rows, jnp.full((16,), k, jnp.int32)], neg)
            return carry

        lax.fori_loop(0, 8, pre, 0)

    def gat(j, b):
        return [
            pltpu.make_async_copy(tabs.at[0].at[idxs_v.at[j]], bs.at[b],
                                  sems.at[b]),
            pltpu.make_async_copy(tabs.at[1].at[idxd_v.at[j]], bd.at[b],
                                  semd.at[b]),
            pltpu.make_async_copy(le8.at[pl.ds(base + j * 128, 128)],
                                  bl.at[b], seml.at[b]),
        ]

    def scat(j, b):
        return pltpu.make_async_copy(
            bo.at[b], out.at[pl.ds(base + j * 128, 128)], semo.at[b])

    scat.add = False
    lof = lo_v[...]

    def process(j, b):
        def rowgrp(v, carry2):
            rows = jnp.full((16,), v * 16, jnp.int32) + iota16
            for q in range(2):
                qf = jnp.full((16,), q, jnp.int32)
                z = (plsc.load_gather(bs.at[b], [rows, qf])
                     + plsc.load_gather(bd.at[b], [rows, qf])
                     + plsc.load_gather(bl.at[b], [rows, qf + lof]))
                z = jnp.maximum(z, 0.2 * z)
                plsc.store_scatter(bo.at[b], [rows, qf], z)
                mx_v[...] = jnp.maximum(mx_v[...], z)
            return carry2

        lax.fori_loop(0, 8, rowgrp, 0)

    _ring(CH, gat, scat, process=process)
    pltpu.sync_copy(mx_v, maxout.at[c, s])


QCH = CH // 2


@_sc_kernel(
    out_type=jax.ShapeDtypeStruct((2, 4, NP, 16), jnp.float32),
    scratch_types=[
        pltpu.VMEM((QCH, 128), jnp.int32),
        pltpu.VMEM((QCH, 128), jnp.int32),
        pltpu.VMEM((7, 128, 16), jnp.float32),
        pltpu.VMEM((7, 128, 8), jnp.float32),
        pltpu.VMEM((16,), jnp.float32),
        pltpu.VMEM_SHARED((NP, 16), jnp.float32),
        pltpu.SemaphoreType.DMA((7,)),
        pltpu.SemaphoreType.DMA((7,)),
        pltpu.SemaphoreType.DMA((7,)),
    ],
)
def _sc_egat(tabw, src3, dst3, logit8, gm16, zeros16, out,
             idxs_v, idxd_v, bufs, lbufs, gm_v, acc_sh, gsems, lsems, ssems):
    c = lax.axis_index("c")
    s = lax.axis_index("s")
    r0 = s * RB
    iota16 = lax.iota(jnp.int32, 16)
    pltpu.sync_copy(gm16, gm_v)
    gm = gm_v[...]

    def one_pass(qp, carry0):
        q = qp // 2
        cg = c * 2 + qp % 2
        qf = jnp.full((16,), 0, jnp.int32) + q
        pltpu.sync_copy(zeros16.at[pl.ds(r0, RB)], acc_sh.at[pl.ds(r0, RB)])
        plsc.subcore_barrier()

        def one_seg(seg, carry1):
            sl = s * 2 + seg // 2
            c0 = (seg % 2) * QCH
            base = sl * SLICE + c0 * 128
            pltpu.sync_copy(src3.at[sl, pl.ds(c0, QCH)], idxs_v)
            pltpu.sync_copy(dst3.at[sl, pl.ds(c0, QCH)], idxd_v)

            def gat(j, b):
                return [
                    pltpu.make_async_copy(
                        tabw.at[q, cg].at[idxs_v.at[j]], bufs.at[b],
                        gsems.at[b]),
                    pltpu.make_async_copy(
                        logit8.at[pl.ds(base + j * 128, 128)], lbufs.at[b],
                        lsems.at[b]),
                ]

            def scat(j, b):
                return pltpu.make_async_copy(
                    bufs.at[b], acc_sh.at[idxd_v.at[j]], ssems.at[b])

            scat.add = True

            def process(j, b):
                buf = bufs.at[b]
                lbuf = lbufs.at[b]

                def rowgrp(v, carry2):
                    for l in range(16):
                        rfull = jnp.full((16,), v * 16 + l, jnp.int32)
                        z = plsc.load_gather(lbuf, [rfull, qf])
                        scale = jnp.exp(z - gm)
                        vals = plsc.load_gather(buf, [rfull, iota16])
                        plsc.store_scatter(buf, [rfull, iota16],
                                           vals * scale)
                    return carry2

                lax.fori_loop(0, 8, rowgrp, 0)

            _ring(QCH, gat, scat, process=process)
            return carry1

        lax.fori_loop(0, 4, one_seg, 0)
        plsc.subcore_barrier()
        pltpu.sync_copy(acc_sh.at[pl.ds(r0, RB)],
                        out.at[q, cg, pl.ds(r0, RB)])
        plsc.subcore_barrier()
        return carry0

    lax.fori_loop(0, 4, one_pass, 0)


def _full(x):
    return pl.BlockSpec(x.shape, lambda i: (0,) * x.ndim)


def _t_edge(ea, We1, be1, ge1, bte1, We2, be2, ge2, bte2, ae1, ae2):
    def body(ea_r, We1_r, be1_r, ge1_r, bte1_r, We2_r, be2_r, ge2_r, bte2_r,
             ae1_r, ae2_r, le_r):
        x = ea_r[...]
        e1 = jnp.maximum((_mm(x, We1_r[...]) + be1_r[...]) * ge1_r[...]
                         + bte1_r[...], 0.0)
        e2 = jnp.maximum((_mm(e1, We2_r[...]) + be2_r[...]) * ge2_r[...]
                         + bte2_r[...], 0.0)
        le_r[...] = jnp.concatenate(
            [_mm(e2, ae1_r[...].T), _mm(e2, ae2_r[...].T),
             jnp.zeros((e2.shape[0], 4), jnp.float32)], axis=1)

    args = (ea, We1, be1, ge1, bte1, We2, be2, ge2, bte2, ae1, ae2)
    return pl.pallas_call(
        body,
        grid=(EG,),
        in_specs=[pl.BlockSpec((EB, NE), lambda i: (i, 0))]
        + [_full(a) for a in args[1:]],
        out_specs=pl.BlockSpec((EB, 8), lambda i: (i, 0)),
        out_shape=jax.ShapeDtypeStruct((EP, 8), jnp.float32),
    )(*args)


def _t_front(dxp, cxp, degp, Wc, bc, Wg0, bg0, Wgc1):
    def body(dx_r, cx_r, deg_r, Wc_r, bc_r, Wg0_r, bg0_r, Wgc1_r,
             xgdc_r, xs1_r):
        x_d = dx_r[...][:, 6:20]
        cx = cx_r[...]
        Wcv = Wc_r[...]
        bcv = bc_r[...]
        parts = [jnp.maximum(_mm(cx[:, 10 * g:10 * g + 10], Wcv) + bcv, 0.0)
                 for g in range(3)]
        xgdc = jnp.concatenate([x_d] + parts, axis=1)
        xgdc_r[...] = xgdc
        xg = jnp.maximum(_mm(xgdc, Wg0_r[...]) + bg0_r[...], 0.0)
        deg = deg_r[0, :, 0] + deg_r[1, :, 0] + 1.0
        dinv = lax.rsqrt(deg)[:, None]
        xs = _mm(xg, Wgc1_r[...]) * dinv
        for g in range(4):
            xs1_r[g] = xs[:, 16 * g:16 * g + 16]

    args = (dxp, cxp, degp, Wc, bc, Wg0, bg0, Wgc1)
    return pl.pallas_call(
        body,
        grid=(NG,),
        in_specs=[pl.BlockSpec((RB, 20), lambda i: (i, 0)),
                  pl.BlockSpec((RB, 30), lambda i: (i, 0)),
                  pl.BlockSpec((2, RB, 8), lambda i: (0, i, 0))]
        + [_full(a) for a in args[3:]],
        out_specs=[pl.BlockSpec((RB, 26), lambda i: (i, 0)),
                   pl.BlockSpec((4, RB, 16), lambda i: (0, i, 0))],
        out_shape=[jax.ShapeDtypeStruct((NP, 26), jnp.float32),
                   jax.ShapeDtypeStruct((4, NP, 16), jnp.float32)],
    )(*args)


def _t_gcn_mid(acc1, xs1, degp, bgc1, Wgc2):
    def body(acc_r, xs_r, deg_r, bgc1_r, Wgc2_r, xg0_r, xs2_r):
        deg = deg_r[0, :, 0] + deg_r[1, :, 0] + 1.0
        dinv = lax.rsqrt(deg)[:, None]
        tot = jnp.concatenate([acc_r[g] + xs_r[g] for g in range(4)],
                              axis=1)
        xg0 = jnp.maximum(tot * dinv + bgc1_r[...], 0.0)
        xg0_r[...] = xg0
        xs = _mm(xg0, Wgc2_r[...]) * dinv
        for g in range(4):
            xs2_r[g] = xs[:, 16 * g:16 * g + 16]

    args = (acc1, xs1, degp, bgc1, Wgc2)
    return pl.pallas_call(
        body,
        grid=(NG,),
        in_specs=[pl.BlockSpec((4, RB, 16), lambda i: (0, i, 0)),
                  pl.BlockSpec((4, RB, 16), lambda i: (0, i, 0)),
                  pl.BlockSpec((2, RB, 8), lambda i: (0, i, 0)),
                  _full(bgc1), _full(Wgc2)],
        out_specs=[pl.BlockSpec((RB, 64), lambda i: (i, 0)),
                   pl.BlockSpec((4, RB, 16), lambda i: (0, i, 0))],
        out_shape=[jax.ShapeDtypeStruct((NP, 64), jnp.float32),
                   jax.ShapeDtypeStruct((4, NP, 16), jnp.float32)],
    )(*args)


def _t_main(acc2, xs2, degp, xg0, xgdc, bgc2, Wr1, br1, gr1, btr1,
            Wr2, br2, gr2, btr2, wcr, bcr, Wf, bf, Ws0, bs0, Wa1, as1, ad1):
    def body(acc_r, xs_r, deg_r, xg0_r, xgdc_r, bgc2_r, Wr1_r, br1_r, gr1_r,
             btr1_r, Wr2_r, br2_r, gr2_r, btr2_r, wcr_r, bcr_r, Wf_r, bf_r,
             Ws0_r, bs0_r, Wa1_r, as1_r, ad1_r, hci_r, xw1_r, tab1_r):
        deg = deg_r[0, :, 0] + deg_r[1, :, 0] + 1.0
        dinv = lax.rsqrt(deg)[:, None]
        tot = jnp.concatenate([acc_r[g] + xs_r[g] for g in range(4)],
                              axis=1)
        xg1 = jnp.maximum(tot * dinv + bgc2_r[...], 0.0)
        x = jnp.concatenate([xgdc_r[...], xg0_r[...] + xg1], axis=1)
        hd = x
        hd = jnp.maximum((_mm(hd, Wr1_r[...]) + br1_r[...]) * gr1_r[...]
                         + btr1_r[...], 0.0) + hd
        hd = jnp.maximum((_mm(hd, Wr2_r[...]) + br2_r[...]) * gr2_r[...]
                         + btr2_r[...], 0.0) + hd
        wcr_v = wcr_r[...]
        bcr_v = bcr_r[...]
        xl = x
        for i in range(2):
            sv = _mm(xl, wcr_v[i][:, None])
            xl = x * sv + bcr_v[i] + xl
        hci_r[...] = jnp.maximum(_mm(hd + xl, Wf_r[...]) + bf_r[...], 0.0)
        xsi = jnp.maximum(_mm(x, Ws0_r[...]) + bs0_r[...], 0.0)
        Wa = Wa1_r[...]
        asv = as1_r[...]
        adv = ad1_r[...]
        scols, dcols = [], []
        for h in range(2):
            xw = _mm(xsi, Wa[h])
            for g in range(4):
                xw1_r[h, g] = xw[:, 16 * g:16 * g + 16]
            scols.append(_mm(xw, asv[h][:, None]))
            dcols.append(_mm(xw, adv[h][:, None]))
        nrow = xsi.shape[0]
        rid = (pl.program_id(0) * nrow
               + lax.broadcasted_iota(jnp.int32, (nrow, 1), 0))
        zpad = jnp.zeros((nrow, 14), jnp.float32)
        tab1_r[0] = jnp.concatenate(scols + [zpad], axis=1)
        ld2 = jnp.where(rid < N, jnp.concatenate(dcols, axis=1), -4e29)
        tab1_r[1] = jnp.concatenate([ld2, zpad], axis=1)

    args = (acc2, xs2, degp, xg0, xgdc, bgc2, Wr1, br1, gr1, btr1,
            Wr2, br2, gr2, btr2, wcr, bcr, Wf, bf, Ws0, bs0, Wa1, as1, ad1)
    RBM = RB // 2
    return pl.pallas_call(
        body,
        grid=(NP // RBM,),
        in_specs=[pl.BlockSpec((4, RBM, 16), lambda i: (0, i, 0)),
                  pl.BlockSpec((4, RBM, 16), lambda i: (0, i, 0)),
                  pl.BlockSpec((2, RBM, 8), lambda i: (0, i, 0)),
                  pl.BlockSpec((RBM, 64), lambda i: (i, 0)),
                  pl.BlockSpec((RBM, 26), lambda i: (i, 0))]
        + [_full(a) for a in args[5:]],
        out_specs=[pl.BlockSpec((RBM, 64), lambda i: (i, 0)),
                   pl.BlockSpec((2, 4, RBM, 16), lambda i: (0, 0, i, 0)),
                   pl.BlockSpec((2, RBM, 16), lambda i: (0, i, 0))],
        out_shape=[jax.ShapeDtypeStruct((N, 64), jnp.float32),
                   jax.ShapeDtypeStruct((2, 4, NP, 16), jnp.float32),
                   jax.ShapeDtypeStruct((2, NP, 16), jnp.float32)],
    )(*args)


def _t_combine(num, denp, bA, Wa2=None, as2=None, ad2=None, x_prev=None,
               make_tables=False, out_n=None):
    def body(*refs):
        if make_tables:
            (num_r, den_r, bA_r, Wa2_r, as2_r, ad2_r,
             xsi_r, xw2_r, tab2_r) = refs
        else:
            (num_r, den_r, bA_r, xp_r, hsi_r) = refs
        d0 = (den_r[0, :, 0] + den_r[1, :, 0] + 1e-16)[:, None]
        d1 = (den_r[0, :, 1] + den_r[1, :, 1] + 1e-16)[:, None]
        cols = []
        for cg in range(4):
            cols.append(0.5 * (num_r[0, cg] / d0 + num_r[1, cg] / d1))
        x = jnp.maximum(jnp.concatenate(cols, axis=1) + bA_r[...], 0.0)
        if make_tables:
            xsi_r[...] = x
            Wa = Wa2_r[...]
            asv = as2_r[...]
            adv = ad2_r[...]
            scols, dcols = [], []
            for h in range(2):
                xw = _mm(x, Wa[h])
                for g in range(4):
                    xw2_r[h, g] = xw[:, 16 * g:16 * g + 16]
                scols.append(_mm(xw, asv[h][:, None]))
                dcols.append(_mm(xw, adv[h][:, None]))
            nrow = x.shape[0]
            rid = (pl.program_id(0) * nrow
                   + lax.broadcasted_iota(jnp.int32, (nrow, 1), 0))
            zpad = jnp.zeros((nrow, 14), jnp.float32)
            tab2_r[0] = jnp.concatenate(scols + [zpad], axis=1)
            ld2 = jnp.where(rid < N, jnp.concatenate(dcols, axis=1), -4e29)
            tab2_r[1] = jnp.concatenate([ld2, zpad], axis=1)
        else:
            hsi_r[...] = x + xp_r[...]

    RBC = RB // 4
    base_specs = [pl.BlockSpec((2, 4, RBC, 16), lambda i: (0, 0, i, 0)),
                  pl.BlockSpec((2, RBC, 8), lambda i: (0, i, 0)),
                  _full(bA)]
    if make_tables:
        args = (num, denp, bA, Wa2, as2, ad2)
        return pl.pallas_call(
            body,
            grid=(NP // RBC,),
            in_specs=base_specs + [_full(Wa2), _full(as2), _full(ad2)],
            out_specs=[pl.BlockSpec((RBC, 64), lambda i: (i, 0)),
                       pl.BlockSpec((2, 4, RBC, 16), lambda i: (0, 0, i, 0)),
                       pl.BlockSpec((2, RBC, 16), lambda i: (0, i, 0))],
            out_shape=[jax.ShapeDtypeStruct((NP, 64), jnp.float32),
                       jax.ShapeDtypeStruct((2, 4, NP, 16), jnp.float32),
                       jax.ShapeDtypeStruct((2, NP, 16), jnp.float32)],
        )(*args)
    args = (num, denp, bA, x_prev)
    return pl.pallas_call(
        body,
        grid=(NP // RBC,),
        in_specs=base_specs + [pl.BlockSpec((RBC, 64), lambda i: (i, 0))],
        out_specs=pl.BlockSpec((RBC, 64), lambda i: (i, 0)),
        out_shape=jax.ShapeDtypeStruct((out_n, 64), jnp.float32),
    )(*args)


def _t_head(h_ci, h_si, t, Wy0, by0, Wy1, by1, Wp0a, bp0a, Wp0b, bp0b,
            Wp1a, bp1a, Wp1b, bp1b, WpT, bpT):
    def body(hc_r, hs_r, t_r, Wy0_r, by0_r, Wy1_r, by1_r, Wp0a_r, bp0a_r,
             Wp0b_r, bp0b_r, Wp1a_r, bp1a_r, Wp1b_r, bp1b_r, WpT_r, bpT_r,
             py_r, pycf_r, py0_r, py1_r, pT_r):
        hc = hc_r[...]
        hs = hs_r[...]
        h = jnp.concatenate([hc, hs], axis=1)

        def smax(z):
            z = z - jnp.max(z, axis=1, keepdims=True)
            ez = jnp.exp(z)
            return ez / jnp.sum(ez, axis=1, keepdims=True)

        a0 = smax(_mm(h, Wy0_r[...]) + by0_r[...])
        a1 = smax(_mm(h, Wy1_r[...]) + by1_r[...])
        py0 = a0[:, :64] * hc + a0[:, 64:] * hs
        py1 = a1[:, :64] * hc + a1[:, 64:] * hs
        py0 = jax.nn.sigmoid(
            _mm(jnp.maximum(_mm(py0, Wp0a_r[...]) + bp0a_r[...], 0.0),
                Wp0b_r[...]) + bp0b_r[...])
        py1 = jax.nn.sigmoid(
            _mm(jnp.maximum(_mm(py1, Wp1a_r[...]) + bp1a_r[...], 0.0),
                Wp1b_r[...]) + bp1b_r[...])
        pT = jax.nn.sigmoid(_mm(hs, WpT_r[...]) + bpT_r[...])
        tv = t_r[...]
        py_r[...] = (1.0 - tv) * py0 + tv * py1
        pycf_r[...] = tv * py0 + (1.0 - tv) * py1
        py0_r[...] = py0
        py1_r[...] = py1
        pT_r[...] = pT

    args = (h_ci, h_si, t, Wy0, by0, Wy1, by1, Wp0a, bp0a, Wp0b, bp0b,
            Wp1a, bp1a, Wp1b, bp1b, WpT, bpT)
    HB = 2000
    o = pl.BlockSpec((HB, 1), lambda i: (i, 0))
    sd = jax.ShapeDtypeStruct((B, 1), jnp.float32)
    return pl.pallas_call(
        body,
        grid=(B // HB,),
        in_specs=[pl.BlockSpec((HB, 64), lambda i: (i, 0)),
                  pl.BlockSpec((HB, 64), lambda i: (i, 0)),
                  pl.BlockSpec((HB, 1), lambda i: (i, 0))]
        + [_full(a) for a in args[3:]],
        out_specs=[o, o, o, o, o],
        out_shape=[sd, sd, sd, sd, sd],
    )(*args)


def kernel(discrete_x, continous_x, edge_index, edge_attr, churn_date, t,
           Wc, bc, We1, be1, ge1, bte1, We2, be2, ge2, bte2,
           Wg0, bg0, Wgc1, bgc1, Wgc2, bgc2,
           Wr1, br1, gr1, btr1, Wr2, br2, gr2, btr2,
           wcr, bcr, Wf, bf, Ws0, bs0,
           Wa1, as1, ad1, ae1, bA1, Wa2, as2, ad2, ae2, bA2,
           Wy0, by0, Wy1, by1,
           Wp0a, bp0a, Wp0b, bp0b, Wp1a, bp1a, Wp1b, bp1b, WpT, bpT):
    f32 = jnp.float32
    src = edge_index[0].astype(jnp.int32)
    dst = edge_index[1].astype(jnp.int32)
    src_p = jnp.concatenate([src, jnp.zeros((EP - E,), jnp.int32)])
    dst_p = jnp.concatenate([dst, jnp.full((EP - E,), TRASH, jnp.int32)])
    src3 = src_p.reshape(32, CH, 128)
    dst3 = dst_p.reshape(32, CH, 128)
    zeros8 = jnp.zeros((NP, 8), f32)
    zeros16 = jnp.zeros((NP, 16), f32)
    ones8 = jnp.ones((128, 8), f32)
    loff1 = jnp.zeros((16,), jnp.int32)
    loff2 = jnp.full((16,), 2, jnp.int32)

    degp = _sc_degree(dst3, zeros8, ones8)
    le8 = _t_edge(edge_attr, We1, be1, ge1, bte1, We2, be2, ge2, bte2,
                  ae1, ae2)
    xgdc, xs1 = _t_front(discrete_x, continous_x, degp, Wc, bc, Wg0, bg0,
                         Wgc1)

    acc1 = _sc_gcn(xs1, src3, dst3, zeros16)
    xg0, xs2 = _t_gcn_mid(acc1, xs1, degp, bgc1, Wgc2)
    acc2 = _sc_gcn(xs2, src3, dst3, zeros16)

    h_ci, xw1, tabs1 = _t_main(
        acc2, xs2, degp, xg0, xgdc, bgc2, Wr1, br1, gr1, btr1,
        Wr2, br2, gr2, btr2, wcr, bcr, Wf, bf, Ws0, bs0, Wa1, as1, ad1)

    logit1, mx1 = _sc_logit(tabs1, src3, dst3, le8, loff1)
    gm16_1 = jnp.broadcast_to(jnp.max(mx1), (16,)).astype(f32)
    den1 = _sc_den(dst3, logit1, gm16_1, zeros8)
    num1 = _sc_egat(xw1, src3, dst3, logit1, gm16_1, zeros16)
    x_si0, xw2, tabs2 = _t_combine(num1, den1, bA1, Wa2=Wa2, as2=as2,
                                   ad2=ad2, make_tables=True)

    logit2, mx2 = _sc_logit(tabs2, src3, dst3, le8, loff2)
    gm16_2 = jnp.broadcast_to(jnp.max(mx2), (16,)).astype(f32)
    den2 = _sc_den(dst3, logit2, gm16_2, zeros8)
    num2 = _sc_egat(xw2, src3, dst3, logit2, gm16_2, zeros16)
    h_si = _t_combine(num2, den2, bA2, x_prev=x_si0, out_n=N)

    pred_y, pred_y_cf, py0, py1, pred_T = _t_head(
        h_ci, h_si, t, Wy0, by0, Wy1, by1, Wp0a, bp0a, Wp0b, bp0b,
        Wp1a, bp1a, Wp1b, bp1b, WpT, bpT)
    return (pred_y, pred_y_cf, py0, py1, pred_T, h_ci, h_si)


# speedup vs baseline: 49.3342x; 1.8597x over previous
import functools

import jax
import jax.numpy as jnp
from jax import lax
from jax.experimental import pallas as pl
from jax.experimental.pallas import tpu as pltpu
from jax.experimental.pallas import tpu_sc as plsc

N = 50000
E = 800000
B = 10000
H = 64
NE = 16
NH1 = 90

NP = 50176
RB = 3136
NG = NP // RB
TRASH = N

EP = 802816
SLICE = 25088
CH = 196
EB = 8192
EG = EP // EB

def _sc_kernel(**kw):
    def deco(fn):
        cache = {}

        def call(*args):
            if 'k' not in cache:
                mesh = plsc.VectorSubcoreMesh(
                    core_axis_name="c", subcore_axis_name="s",
                    num_cores=2, num_subcores=16)
                cache['k'] = pl.kernel(
                    fn, mesh=mesh,
                    compiler_params=pltpu.CompilerParams(
                        use_tc_tiling_on_sc=False,
                        needs_layout_passes=False),
                    **kw)
            return cache['k'](*args)

        return call

    return deco


def _mm(a, b):
    return jnp.dot(a, b, preferred_element_type=jnp.float32)


@_sc_kernel(
    out_type=jax.ShapeDtypeStruct((2, NP, 8), jnp.float32),
    scratch_types=[
        pltpu.VMEM((CH, 128), jnp.int32),
        pltpu.VMEM((128, 8), jnp.float32),
        pltpu.VMEM_SHARED((NP, 8), jnp.float32),
        pltpu.SemaphoreType.DMA,
    ],
)
def _sc_degree(dst3, zeros8, ones8, out, idx_v, ones_v, acc_sh, sem):
    c = lax.axis_index("c")
    s = lax.axis_index("s")
    wid = s * 2 + c
    r0 = s * RB
    pltpu.sync_copy(zeros8.at[pl.ds(r0, RB)], acc_sh.at[pl.ds(r0, RB)])
    pltpu.sync_copy(ones8, ones_v)
    pltpu.sync_copy(dst3.at[wid], idx_v)
    plsc.subcore_barrier()

    def sc(j):
        return pltpu.make_async_copy(ones_v, acc_sh.at[idx_v.at[j]], sem)

    def start(j, carry):
        sc(j).start(add=True)
        return carry

    def drain(j, carry):
        sc(j).wait()
        return carry

    lax.fori_loop(0, CH, start, 0)
    lax.fori_loop(0, CH, drain, 0)
    plsc.subcore_barrier()
    pltpu.sync_copy(acc_sh.at[pl.ds(r0, RB)], out.at[c, pl.ds(r0, RB)])


def _ring(nch, gat, scat, process=None, nbuf=7, prime=5):
    def aslist(d):
        return d if isinstance(d, (list, tuple)) else [d]

    for b in range(prime):
        for d in aslist(gat(b, b)):
            d.start()

    def grp(g, carry):
        for i in range(nbuf):
            j = g * nbuf + i
            for d in aslist(gat(j, i)):
                d.wait()
            if process is not None:
                process(j, i)
            scat(j, i).start(add=scat.add)
            b2 = (i + prime) % nbuf

            @pl.when((j >= 2) & (j + prime < nch))
            def _():
                scat(j - 2, b2).wait()

            @pl.when(j + prime < nch)
            def _():
                for d in aslist(gat(j + prime, b2)):
                    d.start()
        return carry

    lax.fori_loop(0, nch // nbuf, grp, 0)
    for b in range(nbuf):
        scat(nch - nbuf + b, b).wait()


@_sc_kernel(
    out_type=jax.ShapeDtypeStruct((2, NP, 8), jnp.float32),
    scratch_types=[
        pltpu.VMEM((CH, 128), jnp.int32),
        pltpu.VMEM((7, 128, 8), jnp.float32),
        pltpu.VMEM((16,), jnp.float32),
        pltpu.VMEM_SHARED((NP, 8), jnp.float32),
        pltpu.SemaphoreType.DMA((7,)),
        pltpu.SemaphoreType.DMA((7,)),
    ],
)
def _sc_den(dst3, logit8, gm16, zeros8, out, idx_v, bufs, gm_v, acc_sh,
            gsems, ssems):
    c = lax.axis_index("c")
    s = lax.axis_index("s")
    wid = s * 2 + c
    base = wid * SLICE
    r0 = s * RB
    iota16 = lax.iota(jnp.int32, 16)
    pltpu.sync_copy(zeros8.at[pl.ds(r0, RB)], acc_sh.at[pl.ds(r0, RB)])
    pltpu.sync_copy(dst3.at[wid], idx_v)
    pltpu.sync_copy(gm16, gm_v)
    plsc.subcore_barrier()
    gm = gm_v[...]

    def gat(j, b):
        return pltpu.make_async_copy(
            logit8.at[pl.ds(base + j * 128, 128)], bufs.at[b], gsems.at[b])

    def scat(j, b):
        return pltpu.make_async_copy(
            bufs.at[b], acc_sh.at[idx_v.at[j]], ssems.at[b])

    scat.add = True

    def process(j, b):
        def rowgrp(v, carry2):
            rows = jnp.full((16,), v * 16, jnp.int32) + iota16
            for q in range(2):
                qf = jnp.full((16,), q, jnp.int32)
                z = plsc.load_gather(bufs.at[b], [rows, qf])
                plsc.store_scatter(bufs.at[b], [rows, qf], jnp.exp(z - gm))
            for k in range(2, 8):
                plsc.store_scatter(
                    bufs.at[b], [rows, jnp.full((16,), k, jnp.int32)],
                    jnp.zeros((16,), jnp.float32))
            return carry2

        lax.fori_loop(0, 8, rowgrp, 0)

    _ring(CH, gat, scat, process=process)
    plsc.subcore_barrier()
    pltpu.sync_copy(acc_sh.at[pl.ds(r0, RB)], out.at[c, pl.ds(r0, RB)])


@_sc_kernel(
    out_type=jax.ShapeDtypeStruct((4, NP, 16), jnp.float32),
    scratch_types=[
        pltpu.VMEM((CH, 128), jnp.int32),
        pltpu.VMEM((CH, 128), jnp.int32),
        pltpu.VMEM((7, 128, 16), jnp.float32),
        pltpu.VMEM_SHARED((NP, 16), jnp.float32),
        pltpu.SemaphoreType.DMA((7,)),
        pltpu.SemaphoreType.DMA((7,)),
    ],
)
def _sc_gcn(tab4, src3, dst3, zeros16, out, idxs_v, idxd_v, bufs, acc_sh,
            gsems, ssems):
    c = lax.axis_index("c")
    s = lax.axis_index("s")
    r0 = s * RB
    for p in range(2):
        cg = c * 2 + p
        pltpu.sync_copy(zeros16.at[pl.ds(r0, RB)], acc_sh.at[pl.ds(r0, RB)])
        plsc.subcore_barrier()
        for half in range(2):
            sl = s * 2 + half
            pltpu.sync_copy(src3.at[sl], idxs_v)
            pltpu.sync_copy(dst3.at[sl], idxd_v)

            def gat(j, b):
                return pltpu.make_async_copy(
                    tab4.at[cg].at[idxs_v.at[j]], bufs.at[b], gsems.at[b])

            def scat(j, b):
                return pltpu.make_async_copy(
                    bufs.at[b], acc_sh.at[idxd_v.at[j]], ssems.at[b])

            scat.add = True
            _ring(CH, gat, scat)
        plsc.subcore_barrier()
        pltpu.sync_copy(acc_sh.at[pl.ds(r0, RB)], out.at[cg, pl.ds(r0, RB)])
        plsc.subcore_barrier()


@_sc_kernel(
    out_type=[jax.ShapeDtypeStruct((EP, 8), jnp.float32),
              jax.ShapeDtypeStruct((2, 16, 16), jnp.float32)],
    scratch_types=[
        pltpu.VMEM((CH, 128), jnp.int32),
        pltpu.VMEM((CH, 128), jnp.int32),
        pltpu.VMEM((7, 128, 16), jnp.float32),
        pltpu.VMEM((7, 128, 16), jnp.float32),
        pltpu.VMEM((7, 128, 8), jnp.float32),
        pltpu.VMEM((7, 128, 8), jnp.float32),
        pltpu.VMEM((16,), jnp.float32),
        pltpu.VMEM((16,), jnp.int32),
        pltpu.SemaphoreType.DMA((7,)),
        pltpu.SemaphoreType.DMA((7,)),
        pltpu.SemaphoreType.DMA((7,)),
        pltpu.SemaphoreType.DMA((7,)),
    ],
)
def _sc_logit(tabs, src3, dst3, le8, loff16, out, maxout,
              idxs_v, idxd_v, bs, bd, bl, bo, mx_v, lo_v,
              sems, semd, seml, semo):
    c = lax.axis_index("c")
    s = lax.axis_index("s")
    wid = s * 2 + c
    base = wid * SLICE
    iota16 = lax.iota(jnp.int32, 16)
    pltpu.sync_copy(src3.at[wid], idxs_v)
    pltpu.sync_copy(dst3.at[wid], idxd_v)
    pltpu.sync_copy(loff16, lo_v)
    mx_v[...] = jnp.full((16,), -1e30, jnp.float32)
    neg = jnp.full((16,), -1e30, jnp.float32)
    for b in range(7):
        def pre(v, carry):
            rows = jnp.full((16,), v * 16, jnp.int32) + iota16
            for k in range(2, 8):
                plsc.store_scatter(
                    bo.at[b], [rows, jnp.full((16,), k, jnp.int32)], neg)
            return carry

        lax.fori_loop(0, 8, pre, 0)

    def gat(j, b):
        return [
            pltpu.make_async_copy(tabs.at[0].at[idxs_v.at[j]], bs.at[b],
                                  sems.at[b]),
            pltpu.make_async_copy(tabs.at[1].at[idxd_v.at[j]], bd.at[b],
                                  semd.at[b]),
            pltpu.make_async_copy(le8.at[pl.ds(base + j * 128, 128)],
                                  bl.at[b], seml.at[b]),
        ]

    def scat(j, b):
        return pltpu.make_async_copy(
            bo.at[b], out.at[pl.ds(base + j * 128, 128)], semo.at[b])

    scat.add = False
    lof = lo_v[...]

    def process(j, b):
        def rowgrp(v, carry2):
            rows = jnp.full((16,), v * 16, jnp.int32) + iota16
            for q in range(2):
                qf = jnp.full((16,), q, jnp.int32)
                z = (plsc.load_gather(bs.at[b], [rows, qf])
                     + plsc.load_gather(bd.at[b], [rows, qf])
                     + plsc.load_gather(bl.at[b], [rows, qf + lof]))
                z = jnp.maximum(z, 0.2 * z)
                plsc.store_scatter(bo.at[b], [rows, qf], z)
                mx_v[...] = jnp.maximum(mx_v[...], z)
            return carry2

        lax.fori_loop(0, 8, rowgrp, 0)

    _ring(CH, gat, scat, process=process)
    pltpu.sync_copy(mx_v, maxout.at[c, s])


ECH = 49


@_sc_kernel(
    out_type=jax.ShapeDtypeStruct((4, NP, 16), jnp.float32),
    scratch_types=[
        pltpu.VMEM((ECH, 128), jnp.int32),
        pltpu.VMEM((ECH, 128), jnp.int32),
        pltpu.VMEM((7, 128, 32), jnp.float32),
        pltpu.VMEM((7, 128, 8), jnp.float32),
        pltpu.VMEM((7, 128, 8), jnp.float32),
        pltpu.VMEM((7, 128, 16), jnp.float32),
        pltpu.VMEM((16,), jnp.float32),
        pltpu.VMEM_SHARED((NP, 16), jnp.float32),
        pltpu.SemaphoreType.DMA((7,)),
        pltpu.SemaphoreType.DMA((7,)),
        pltpu.SemaphoreType.DMA((7,)),
        pltpu.SemaphoreType.DMA((7,)),
    ],
)
def _sc_egat(tabw, src3, dst3, logit8, rden8, gm16, zeros16, out,
             idxs_v, idxd_v, gbufs, lbufs, rbufs, sbufs, gm_v, acc_sh,
             gsems, lsems, rsems, ssems):
    c = lax.axis_index("c")
    s = lax.axis_index("s")
    r0 = s * RB
    iota16 = lax.iota(jnp.int32, 16)
    iotahi = iota16 + 16
    pltpu.sync_copy(gm16, gm_v)
    gm = gm_v[...]
    c0f = jnp.zeros((16,), jnp.int32)
    c1f = c0f + 1

    def one_pass(p, carry0):
        cg = c * 2 + p
        pltpu.sync_copy(zeros16.at[pl.ds(r0, RB)], acc_sh.at[pl.ds(r0, RB)])
        plsc.subcore_barrier()

        def one_seg(seg, carry1):
            sl = s * 2 + seg // 4
            ch0 = (seg % 4) * ECH
            base = sl * SLICE + ch0 * 128
            pltpu.sync_copy(src3.at[sl, pl.ds(ch0, ECH)], idxs_v)
            pltpu.sync_copy(dst3.at[sl, pl.ds(ch0, ECH)], idxd_v)

            def gat(j, b):
                return [
                    pltpu.make_async_copy(
                        tabw.at[cg].at[idxs_v.at[j]], gbufs.at[b],
                        gsems.at[b]),
                    pltpu.make_async_copy(
                        logit8.at[pl.ds(base + j * 128, 128)], lbufs.at[b],
                        lsems.at[b]),
                    pltpu.make_async_copy(
                        rden8.at[idxd_v.at[j]], rbufs.at[b], rsems.at[b]),
                ]

            def scat(j, b):
                return pltpu.make_async_copy(
                    sbufs.at[b], acc_sh.at[idxd_v.at[j]], ssems.at[b])

            scat.add = True

            dnums = lax.GatherDimensionNumbers(
                offset_dims=(), collapsed_slice_dims=(0,),
                start_index_map=(0,))

            def bcast(vec, l):
                return lax.gather(
                    vec, jnp.full((16, 1), l, jnp.int32), dnums, (1,),
                    mode=lax.GatherScatterMode.PROMISE_IN_BOUNDS)

            def process(j, b):
                gbuf = gbufs.at[b]
                lbuf = lbufs.at[b]
                rbuf = rbufs.at[b]
                sbuf = sbufs.at[b]

                def rowgrp(v, carry2):
                    for vv2 in range(2):
                        v0 = v * 32 + vv2 * 16
                        rows = jnp.full((16,), v0, jnp.int32) + iota16
                        exw = []
                        for q in range(2):
                            qf = c0f + q
                            z = plsc.load_gather(lbuf, [rows, qf])
                            rd = plsc.load_gather(rbuf, [rows, qf])
                            exw.append(jnp.exp(z - gm) * rd)
                        for l in range(16):
                            rr = v0 + l
                            a = gbuf[rr, pl.ds(0, 16)]
                            bb = gbuf[rr, pl.ds(16, 16)]
                            sbuf[rr, pl.ds(0, 16)] = (
                                a * bcast(exw[0], l) + bb * bcast(exw[1], l))
                    return carry2

                lax.fori_loop(0, 4, rowgrp, 0)

            _ring(ECH, gat, scat, process=process)
            return carry1

        lax.fori_loop(0, 8, one_seg, 0)
        plsc.subcore_barrier()
        pltpu.sync_copy(acc_sh.at[pl.ds(r0, RB)],
                        out.at[cg, pl.ds(r0, RB)])
        plsc.subcore_barrier()
        return carry0

    lax.fori_loop(0, 2, one_pass, 0)


def _full(x):
    return pl.BlockSpec(x.shape, lambda i: (0,) * x.ndim)


def _t_edge(ea, We1, be1, ge1, bte1, We2, be2, ge2, bte2, ae1, ae2):
    def body(ea_r, We1_r, be1_r, ge1_r, bte1_r, We2_r, be2_r, ge2_r, bte2_r,
             ae1_r, ae2_r, le_r):
        x = ea_r[...]
        e1 = jnp.maximum((_mm(x, We1_r[...]) + be1_r[...]) * ge1_r[...]
                         + bte1_r[...], 0.0)
        e2 = jnp.maximum((_mm(e1, We2_r[...]) + be2_r[...]) * ge2_r[...]
                         + bte2_r[...], 0.0)
        le_r[...] = jnp.concatenate(
            [_mm(e2, ae1_r[...].T), _mm(e2, ae2_r[...].T),
             jnp.zeros((e2.shape[0], 4), jnp.float32)], axis=1)

    args = (ea, We1, be1, ge1, bte1, We2, be2, ge2, bte2, ae1, ae2)
    return pl.pallas_call(
        body,
        grid=(EG,),
        in_specs=[pl.BlockSpec((EB, NE), lambda i: (i, 0))]
        + [_full(a) for a in args[1:]],
        out_specs=pl.BlockSpec((EB, 8), lambda i: (i, 0)),
        out_shape=jax.ShapeDtypeStruct((EP, 8), jnp.float32),
    )(*args)


def _t_front(dxp, cxp, degp, Wc, bc, Wg0, bg0, Wgc1):
    def body(dx_r, cx_r, deg_r, Wc_r, bc_r, Wg0_r, bg0_r, Wgc1_r,
             xgdc_r, xs1_r):
        x_d = dx_r[...][:, 6:20]
        cx = cx_r[...]
        Wcv = Wc_r[...]
        bcv = bc_r[...]
        parts = [jnp.maximum(_mm(cx[:, 10 * g:10 * g + 10], Wcv) + bcv, 0.0)
                 for g in range(3)]
        xgdc = jnp.concatenate([x_d] + parts, axis=1)
        xgdc_r[...] = xgdc
        xg = jnp.maximum(_mm(xgdc, Wg0_r[...]) + bg0_r[...], 0.0)
        deg = deg_r[0, :, 0] + deg_r[1, :, 0] + 1.0
        dinv = lax.rsqrt(deg)[:, None]
        xs = _mm(xg, Wgc1_r[...]) * dinv
        for g in range(4):
            xs1_r[g] = xs[:, 16 * g:16 * g + 16]

    args = (dxp, cxp, degp, Wc, bc, Wg0, bg0, Wgc1)
    return pl.pallas_call(
        body,
        grid=(NG,),
        in_specs=[pl.BlockSpec((RB, 20), lambda i: (i, 0)),
                  pl.BlockSpec((RB, 30), lambda i: (i, 0)),
                  pl.BlockSpec((2, RB, 8), lambda i: (0, i, 0))]
        + [_full(a) for a in args[3:]],
        out_specs=[pl.BlockSpec((RB, 26), lambda i: (i, 0)),
                   pl.BlockSpec((4, RB, 16), lambda i: (0, i, 0))],
        out_shape=[jax.ShapeDtypeStruct((NP, 26), jnp.float32),
                   jax.ShapeDtypeStruct((4, NP, 16), jnp.float32)],
    )(*args)


def _t_gcn_mid(acc1, xs1, degp, bgc1, Wgc2):
    def body(acc_r, xs_r, deg_r, bgc1_r, Wgc2_r, xg0_r, xs2_r):
        deg = deg_r[0, :, 0] + deg_r[1, :, 0] + 1.0
        dinv = lax.rsqrt(deg)[:, None]
        tot = jnp.concatenate([acc_r[g] + xs_r[g] for g in range(4)],
                              axis=1)
        xg0 = jnp.maximum(tot * dinv + bgc1_r[...], 0.0)
        xg0_r[...] = xg0
        xs = _mm(xg0, Wgc2_r[...]) * dinv
        for g in range(4):
            xs2_r[g] = xs[:, 16 * g:16 * g + 16]

    args = (acc1, xs1, degp, bgc1, Wgc2)
    return pl.pallas_call(
        body,
        grid=(NG,),
        in_specs=[pl.BlockSpec((4, RB, 16), lambda i: (0, i, 0)),
                  pl.BlockSpec((4, RB, 16), lambda i: (0, i, 0)),
                  pl.BlockSpec((2, RB, 8), lambda i: (0, i, 0)),
                  _full(bgc1), _full(Wgc2)],
        out_specs=[pl.BlockSpec((RB, 64), lambda i: (i, 0)),
                   pl.BlockSpec((4, RB, 16), lambda i: (0, i, 0))],
        out_shape=[jax.ShapeDtypeStruct((NP, 64), jnp.float32),
                   jax.ShapeDtypeStruct((4, NP, 16), jnp.float32)],
    )(*args)


def _t_main(acc2, xs2, degp, xg0, xgdc, bgc2, Wr1, br1, gr1, btr1,
            Wr2, br2, gr2, btr2, wcr, bcr, Wf, bf, Ws0, bs0, Wa1, as1, ad1):
    def body(acc_r, xs_r, deg_r, xg0_r, xgdc_r, bgc2_r, Wr1_r, br1_r, gr1_r,
             btr1_r, Wr2_r, br2_r, gr2_r, btr2_r, wcr_r, bcr_r, Wf_r, bf_r,
             Ws0_r, bs0_r, Wa1_r, as1_r, ad1_r, hci_r, xw1_r, tab1_r):
        deg = deg_r[0, :, 0] + deg_r[1, :, 0] + 1.0
        dinv = lax.rsqrt(deg)[:, None]
        tot = jnp.concatenate([acc_r[g] + xs_r[g] for g in range(4)],
                              axis=1)
        xg1 = jnp.maximum(tot * dinv + bgc2_r[...], 0.0)
        x = jnp.concatenate([xgdc_r[...], xg0_r[...] + xg1], axis=1)
        hd = x
        hd = jnp.maximum((_mm(hd, Wr1_r[...]) + br1_r[...]) * gr1_r[...]
                         + btr1_r[...], 0.0) + hd
        hd = jnp.maximum((_mm(hd, Wr2_r[...]) + br2_r[...]) * gr2_r[...]
                         + btr2_r[...], 0.0) + hd
        wcr_v = wcr_r[...]
        bcr_v = bcr_r[...]
        xl = x
        for i in range(2):
            sv = _mm(xl, wcr_v[i][:, None])
            xl = x * sv + bcr_v[i] + xl
        hci_r[...] = jnp.maximum(_mm(hd + xl, Wf_r[...]) + bf_r[...], 0.0)
        xsi = jnp.maximum(_mm(x, Ws0_r[...]) + bs0_r[...], 0.0)
        Wa = Wa1_r[...]
        asv = as1_r[...]
        adv = ad1_r[...]
        scols, dcols = [], []
        xws = []
        for h in range(2):
            xw = _mm(xsi, Wa[h])
            xws.append(xw)
            scols.append(_mm(xw, asv[h][:, None]))
            dcols.append(_mm(xw, adv[h][:, None]))
        zw = jnp.zeros((xws[0].shape[0], 96), jnp.float32)
        for g in range(4):
            xw1_r[g] = jnp.concatenate(
                [xws[0][:, 16 * g:16 * g + 16],
                 xws[1][:, 16 * g:16 * g + 16], zw], axis=1)
        nrow = xsi.shape[0]
        rid = (pl.program_id(0) * nrow
               + lax.broadcasted_iota(jnp.int32, (nrow, 1), 0))
        zpad = jnp.zeros((nrow, 14), jnp.float32)
        tab1_r[0] = jnp.concatenate(scols + [zpad], axis=1)
        ld2 = jnp.where(rid < N, jnp.concatenate(dcols, axis=1), -4e29)
        tab1_r[1] = jnp.concatenate([ld2, zpad], axis=1)

    args = (acc2, xs2, degp, xg0, xgdc, bgc2, Wr1, br1, gr1, btr1,
            Wr2, br2, gr2, btr2, wcr, bcr, Wf, bf, Ws0, bs0, Wa1, as1, ad1)
    RBM = RB // 2
    return pl.pallas_call(
        body,
        grid=(NP // RBM,),
        in_specs=[pl.BlockSpec((4, RBM, 16), lambda i: (0, i, 0)),
                  pl.BlockSpec((4, RBM, 16), lambda i: (0, i, 0)),
                  pl.BlockSpec((2, RBM, 8), lambda i: (0, i, 0)),
                  pl.BlockSpec((RBM, 64), lambda i: (i, 0)),
                  pl.BlockSpec((RBM, 26), lambda i: (i, 0))]
        + [_full(a) for a in args[5:]],
        out_specs=[pl.BlockSpec((RBM, 64), lambda i: (i, 0)),
                   pl.BlockSpec((4, RBM, 128), lambda i: (0, i, 0)),
                   pl.BlockSpec((2, RBM, 16), lambda i: (0, i, 0))],
        out_shape=[jax.ShapeDtypeStruct((N, 64), jnp.float32),
                   jax.ShapeDtypeStruct((4, NP, 128), jnp.float32),
                   jax.ShapeDtypeStruct((2, NP, 16), jnp.float32)],
    )(*args)


def _t_rden(denp):
    def body(den_r, out_r):
        d0 = (den_r[0, :, 0] + den_r[1, :, 0] + 1e-16)[:, None]
        d1 = (den_r[0, :, 1] + den_r[1, :, 1] + 1e-16)[:, None]
        nrow = d0.shape[0]
        out_r[...] = jnp.concatenate(
            [1.0 / d0, 1.0 / d1, jnp.zeros((nrow, 6), jnp.float32)], axis=1)

    return pl.pallas_call(
        body,
        grid=(NG,),
        in_specs=[pl.BlockSpec((2, RB, 8), lambda i: (0, i, 0))],
        out_specs=pl.BlockSpec((RB, 8), lambda i: (i, 0)),
        out_shape=jax.ShapeDtypeStruct((NP, 8), jnp.float32),
    )(denp)


def _t_combine(num, bA, Wa2=None, as2=None, ad2=None, x_prev=None,
               make_tables=False, out_n=None):
    def body(*refs):
        if make_tables:
            (num_r, bA_r, Wa2_r, as2_r, ad2_r,
             xsi_r, xw2_r, tab2_r) = refs
        else:
            (num_r, bA_r, xp_r, hsi_r) = refs
        cols = [0.5 * num_r[cg] for cg in range(4)]
        x = jnp.maximum(jnp.concatenate(cols, axis=1) + bA_r[...], 0.0)
        if make_tables:
            xsi_r[...] = x
            Wa = Wa2_r[...]
            asv = as2_r[...]
            adv = ad2_r[...]
            scols, dcols = [], []
            xws = []
            for h in range(2):
                xw = _mm(x, Wa[h])
                xws.append(xw)
                scols.append(_mm(xw, asv[h][:, None]))
                dcols.append(_mm(xw, adv[h][:, None]))
            zw = jnp.zeros((xws[0].shape[0], 96), jnp.float32)
            for g in range(4):
                xw2_r[g] = jnp.concatenate(
                    [xws[0][:, 16 * g:16 * g + 16],
                     xws[1][:, 16 * g:16 * g + 16], zw], axis=1)
            nrow = x.shape[0]
            rid = (pl.program_id(0) * nrow
                   + lax.broadcasted_iota(jnp.int32, (nrow, 1), 0))
            zpad = jnp.zeros((nrow, 14), jnp.float32)
            tab2_r[0] = jnp.concatenate(scols + [zpad], axis=1)
            ld2 = jnp.where(rid < N, jnp.concatenate(dcols, axis=1), -4e29)
            tab2_r[1] = jnp.concatenate([ld2, zpad], axis=1)
        else:
            hsi_r[...] = x + xp_r[...]

    RBC = RB // 4
    base_specs = [pl.BlockSpec((4, RBC, 16), lambda i: (0, i, 0)),
                  _full(bA)]
    if make_tables:
        args = (num, bA, Wa2, as2, ad2)
        return pl.pallas_call(
            body,
            grid=(NP // RBC,),
            in_specs=base_specs + [_full(Wa2), _full(as2), _full(ad2)],
            out_specs=[pl.BlockSpec((RBC, 64), lambda i: (i, 0)),
                       pl.BlockSpec((4, RBC, 128), lambda i: (0, i, 0)),
                       pl.BlockSpec((2, RBC, 16), lambda i: (0, i, 0))],
            out_shape=[jax.ShapeDtypeStruct((NP, 64), jnp.float32),
                       jax.ShapeDtypeStruct((4, NP, 128), jnp.float32),
                       jax.ShapeDtypeStruct((2, NP, 16), jnp.float32)],
        )(*args)
    args = (num, bA, x_prev)
    return pl.pallas_call(
        body,
        grid=(NP // RBC,),
        in_specs=base_specs + [pl.BlockSpec((RBC, 64), lambda i: (i, 0))],
        out_specs=pl.BlockSpec((RBC, 64), lambda i: (i, 0)),
        out_shape=jax.ShapeDtypeStruct((out_n, 64), jnp.float32),
    )(*args)


def _t_head(h_ci, h_si, t, Wy0, by0, Wy1, by1, Wp0a, bp0a, Wp0b, bp0b,
            Wp1a, bp1a, Wp1b, bp1b, WpT, bpT):
    def body(hc_r, hs_r, t_r, Wy0_r, by0_r, Wy1_r, by1_r, Wp0a_r, bp0a_r,
             Wp0b_r, bp0b_r, Wp1a_r, bp1a_r, Wp1b_r, bp1b_r, WpT_r, bpT_r,
             py_r, pycf_r, py0_r, py1_r, pT_r):
        hc = hc_r[...]
        hs = hs_r[...]
        h = jnp.concatenate([hc, hs], axis=1)

        def smax(z):
            z = z - jnp.max(z, axis=1, keepdims=True)
            ez = jnp.exp(z)
            return ez / jnp.sum(ez, axis=1, keepdims=True)

        a0 = smax(_mm(h, Wy0_r[...]) + by0_r[...])
        a1 = smax(_mm(h, Wy1_r[...]) + by1_r[...])
        py0 = a0[:, :64] * hc + a0[:, 64:] * hs
        py1 = a1[:, :64] * hc + a1[:, 64:] * hs
        py0 = jax.nn.sigmoid(
            _mm(jnp.maximum(_mm(py0, Wp0a_r[...]) + bp0a_r[...], 0.0),
                Wp0b_r[...]) + bp0b_r[...])
        py1 = jax.nn.sigmoid(
            _mm(jnp.maximum(_mm(py1, Wp1a_r[...]) + bp1a_r[...], 0.0),
                Wp1b_r[...]) + bp1b_r[...])
        pT = jax.nn.sigmoid(_mm(hs, WpT_r[...]) + bpT_r[...])
        tv = t_r[...]
        py_r[...] = (1.0 - tv) * py0 + tv * py1
        pycf_r[...] = tv * py0 + (1.0 - tv) * py1
        py0_r[...] = py0
        py1_r[...] = py1
        pT_r[...] = pT

    args = (h_ci, h_si, t, Wy0, by0, Wy1, by1, Wp0a, bp0a, Wp0b, bp0b,
            Wp1a, bp1a, Wp1b, bp1b, WpT, bpT)
    HB = 2000
    o = pl.BlockSpec((HB, 1), lambda i: (i, 0))
    sd = jax.ShapeDtypeStruct((B, 1), jnp.float32)
    return pl.pallas_call(
        body,
        grid=(B // HB,),
        in_specs=[pl.BlockSpec((HB, 64), lambda i: (i, 0)),
                  pl.BlockSpec((HB, 64), lambda i: (i, 0)),
                  pl.BlockSpec((HB, 1), lambda i: (i, 0))]
        + [_full(a) for a in args[3:]],
        out_specs=[o, o, o, o, o],
        out_shape=[sd, sd, sd, sd, sd],
    )(*args)


def kernel(discrete_x, continous_x, edge_index, edge_attr, churn_date, t,
           Wc, bc, We1, be1, ge1, bte1, We2, be2, ge2, bte2,
           Wg0, bg0, Wgc1, bgc1, Wgc2, bgc2,
           Wr1, br1, gr1, btr1, Wr2, br2, gr2, btr2,
           wcr, bcr, Wf, bf, Ws0, bs0,
           Wa1, as1, ad1, ae1, bA1, Wa2, as2, ad2, ae2, bA2,
           Wy0, by0, Wy1, by1,
           Wp0a, bp0a, Wp0b, bp0b, Wp1a, bp1a, Wp1b, bp1b, WpT, bpT):
    f32 = jnp.float32
    src = edge_index[0].astype(jnp.int32)
    dst = edge_index[1].astype(jnp.int32)
    src_p = jnp.concatenate([src, jnp.zeros((EP - E,), jnp.int32)])
    dst_p = jnp.concatenate([dst, jnp.full((EP - E,), TRASH, jnp.int32)])
    src3 = src_p.reshape(32, CH, 128)
    dst3 = dst_p.reshape(32, CH, 128)
    src3q = (src_p * 4).reshape(32, CH, 128)
    zeros8 = jnp.zeros((NP, 8), f32)
    zeros16 = jnp.zeros((NP, 16), f32)
    ones8 = jnp.ones((128, 8), f32)
    loff1 = jnp.zeros((16,), jnp.int32)
    loff2 = jnp.full((16,), 2, jnp.int32)

    degp = _sc_degree(dst3, zeros8, ones8)
    le8 = _t_edge(edge_attr, We1, be1, ge1, bte1, We2, be2, ge2, bte2,
                  ae1, ae2)
    xgdc, xs1 = _t_front(discrete_x, continous_x, degp, Wc, bc, Wg0, bg0,
                         Wgc1)

    acc1 = _sc_gcn(xs1, src3, dst3, zeros16)
    xg0, xs2 = _t_gcn_mid(acc1, xs1, degp, bgc1, Wgc2)
    acc2 = _sc_gcn(xs2, src3, dst3, zeros16)

    h_ci, xw1, tabs1 = _t_main(
        acc2, xs2, degp, xg0, xgdc, bgc2, Wr1, br1, gr1, btr1,
        Wr2, br2, gr2, btr2, wcr, bcr, Wf, bf, Ws0, bs0, Wa1, as1, ad1)

    logit1, mx1 = _sc_logit(tabs1, src3, dst3, le8, loff1)
    gm16_1 = jnp.broadcast_to(jnp.max(mx1), (16,)).astype(f32)
    den1 = _sc_den(dst3, logit1, gm16_1, zeros8)
    rden1 = _t_rden(den1)
    xw1v = xw1.reshape(4, 4 * NP, 32)
    num1 = _sc_egat(xw1v, src3q, dst3, logit1, rden1, gm16_1, zeros16)
    x_si0, xw2, tabs2 = _t_combine(num1, bA1, Wa2=Wa2, as2=as2,
                                   ad2=ad2, make_tables=True)

    logit2, mx2 = _sc_logit(tabs2, src3, dst3, le8, loff2)
    gm16_2 = jnp.broadcast_to(jnp.max(mx2), (16,)).astype(f32)
    den2 = _sc_den(dst3, logit2, gm16_2, zeros8)
    rden2 = _t_rden(den2)
    xw2v = xw2.reshape(4, 4 * NP, 32)
    num2 = _sc_egat(xw2v, src3q, dst3, logit2, rden2, gm16_2, zeros16)
    h_si = _t_combine(num2, bA2, x_prev=x_si0, out_n=N)

    pred_y, pred_y_cf, py0, py1, pred_T = _t_head(
        h_ci, h_si, t, Wy0, by0, Wy1, by1, Wp0a, bp0a, Wp0b, bp0b,
        Wp1a, bp1a, Wp1b, bp1b, WpT, bpT)
    return (pred_y, pred_y_cf, py0, py1, pred_T, h_ci, h_si)


# speedup vs baseline: 49.3398x; 1.0001x over previous
import jax
import jax.numpy as jnp
from jax import lax
from jax.experimental import pallas as pl
from jax.experimental.pallas import tpu as pltpu
from jax.experimental.pallas import tpu_sc as plsc

N = 50000
E = 800000
B = 10000
H = 64
NE = 16
NH1 = 90

NP = 50176
RB = 3136
NG = NP // RB
TRASH = N

EP = 802816
SLICE = 25088
CH = 196
EB = 8192
EG = EP // EB

def _sc_kernel(**kw):
    def deco(fn):
        cache = {}

        def call(*args):
            if 'k' not in cache:
                mesh = plsc.VectorSubcoreMesh(
                    core_axis_name="c", subcore_axis_name="s",
                    num_cores=2, num_subcores=16)
                cache['k'] = pl.kernel(
                    fn, mesh=mesh,
                    compiler_params=pltpu.CompilerParams(
                        use_tc_tiling_on_sc=False,
                        needs_layout_passes=False),
                    **kw)
            return cache['k'](*args)

        return call

    return deco


def _mm(a, b):
    return jnp.dot(a, b, preferred_element_type=jnp.float32)


@_sc_kernel(
    out_type=jax.ShapeDtypeStruct((2, NP, 8), jnp.float32),
    scratch_types=[
        pltpu.VMEM((CH, 128), jnp.int32),
        pltpu.VMEM((128, 8), jnp.float32),
        pltpu.VMEM_SHARED((NP, 8), jnp.float32),
        pltpu.SemaphoreType.DMA,
    ],
)
def _sc_degree(dst3, zeros8, ones8, out, idx_v, ones_v, acc_sh, sem):
    c = lax.axis_index("c")
    s = lax.axis_index("s")
    wid = s * 2 + c
    r0 = s * RB
    pltpu.sync_copy(zeros8.at[pl.ds(r0, RB)], acc_sh.at[pl.ds(r0, RB)])
    pltpu.sync_copy(ones8, ones_v)
    pltpu.sync_copy(dst3.at[wid], idx_v)
    plsc.subcore_barrier()

    def sc(j):
        return pltpu.make_async_copy(ones_v, acc_sh.at[idx_v.at[j]], sem)

    def start(j, carry):
        sc(j).start(add=True)
        return carry

    def drain(j, carry):
        sc(j).wait()
        return carry

    lax.fori_loop(0, CH, start, 0)
    lax.fori_loop(0, CH, drain, 0)
    plsc.subcore_barrier()
    pltpu.sync_copy(acc_sh.at[pl.ds(r0, RB)], out.at[c, pl.ds(r0, RB)])


def _ring(nch, gat, scat, process=None, nbuf=7, prime=5):
    def aslist(d):
        return d if isinstance(d, (list, tuple)) else [d]

    for b in range(prime):
        for d in aslist(gat(b, b)):
            d.start()

    def grp(g, carry):
        for i in range(nbuf):
            j = g * nbuf + i
            for d in aslist(gat(j, i)):
                d.wait()
            if process is not None:
                process(j, i)
            scat(j, i).start(add=scat.add)
            b2 = (i + prime) % nbuf

            @pl.when((j >= 2) & (j + prime < nch))
            def _():
                scat(j - 2, b2).wait()

            @pl.when(j + prime < nch)
            def _():
                for d in aslist(gat(j + prime, b2)):
                    d.start()
        return carry

    lax.fori_loop(0, nch // nbuf, grp, 0)
    for b in range(nbuf):
        scat(nch - nbuf + b, b).wait()


@_sc_kernel(
    out_type=jax.ShapeDtypeStruct((2, NP, 8), jnp.float32),
    scratch_types=[
        pltpu.VMEM((CH, 128), jnp.int32),
        pltpu.VMEM((7, 128, 8), jnp.float32),
        pltpu.VMEM((16,), jnp.float32),
        pltpu.VMEM_SHARED((NP, 8), jnp.float32),
        pltpu.SemaphoreType.DMA((7,)),
        pltpu.SemaphoreType.DMA((7,)),
    ],
)
def _sc_den(dst3, logit8, gm16, zeros8, out, idx_v, bufs, gm_v, acc_sh,
            gsems, ssems):
    c = lax.axis_index("c")
    s = lax.axis_index("s")
    wid = s * 2 + c
    base = wid * SLICE
    r0 = s * RB
    iota16 = lax.iota(jnp.int32, 16)
    pltpu.sync_copy(zeros8.at[pl.ds(r0, RB)], acc_sh.at[pl.ds(r0, RB)])
    pltpu.sync_copy(dst3.at[wid], idx_v)
    pltpu.sync_copy(gm16, gm_v)
    plsc.subcore_barrier()
    gm = gm_v[...]

    def gat(j, b):
        return pltpu.make_async_copy(
            logit8.at[pl.ds(base + j * 128, 128)], bufs.at[b], gsems.at[b])

    def scat(j, b):
        return pltpu.make_async_copy(
            bufs.at[b], acc_sh.at[idx_v.at[j]], ssems.at[b])

    scat.add = True

    def process(j, b):
        def rowgrp(v, carry2):
            rows = jnp.full((16,), v * 16, jnp.int32) + iota16
            for q in range(2):
                qf = jnp.full((16,), q, jnp.int32)
                z = plsc.load_gather(bufs.at[b], [rows, qf])
                plsc.store_scatter(bufs.at[b], [rows, qf], jnp.exp(z - gm))
            for k in range(2, 8):
                plsc.store_scatter(
                    bufs.at[b], [rows, jnp.full((16,), k, jnp.int32)],
                    jnp.zeros((16,), jnp.float32))
            return carry2

        lax.fori_loop(0, 8, rowgrp, 0)

    _ring(CH, gat, scat, process=process)
    plsc.subcore_barrier()
    pltpu.sync_copy(acc_sh.at[pl.ds(r0, RB)], out.at[c, pl.ds(r0, RB)])


@_sc_kernel(
    out_type=jax.ShapeDtypeStruct((4, NP, 16), jnp.float32),
    scratch_types=[
        pltpu.VMEM((CH, 128), jnp.int32),
        pltpu.VMEM((CH, 128), jnp.int32),
        pltpu.VMEM((7, 128, 16), jnp.float32),
        pltpu.VMEM_SHARED((NP, 16), jnp.float32),
        pltpu.SemaphoreType.DMA((7,)),
        pltpu.SemaphoreType.DMA((7,)),
    ],
)
def _sc_gcn(tab4, src3, dst3, zeros16, out, idxs_v, idxd_v, bufs, acc_sh,
            gsems, ssems):
    c = lax.axis_index("c")
    s = lax.axis_index("s")
    r0 = s * RB
    for p in range(2):
        cg = c * 2 + p
        pltpu.sync_copy(zeros16.at[pl.ds(r0, RB)], acc_sh.at[pl.ds(r0, RB)])
        plsc.subcore_barrier()
        for half in range(2):
            sl = s * 2 + half
            pltpu.sync_copy(src3.at[sl], idxs_v)
            pltpu.sync_copy(dst3.at[sl], idxd_v)

            def gat(j, b):
                return pltpu.make_async_copy(
                    tab4.at[cg].at[idxs_v.at[j]], bufs.at[b], gsems.at[b])

            def scat(j, b):
                return pltpu.make_async_copy(
                    bufs.at[b], acc_sh.at[idxd_v.at[j]], ssems.at[b])

            scat.add = True
            _ring(CH, gat, scat)
        plsc.subcore_barrier()
        pltpu.sync_copy(acc_sh.at[pl.ds(r0, RB)], out.at[cg, pl.ds(r0, RB)])
        plsc.subcore_barrier()


@_sc_kernel(
    out_type=[jax.ShapeDtypeStruct((EP, 8), jnp.float32),
              jax.ShapeDtypeStruct((2, 16, 16), jnp.float32)],
    scratch_types=[
        pltpu.VMEM((CH, 128), jnp.int32),
        pltpu.VMEM((CH, 128), jnp.int32),
        pltpu.VMEM((7, 128, 16), jnp.float32),
        pltpu.VMEM((7, 128, 16), jnp.float32),
        pltpu.VMEM((7, 128, 8), jnp.float32),
        pltpu.VMEM((7, 128, 8), jnp.float32),
        pltpu.VMEM((16,), jnp.float32),
        pltpu.VMEM((16,), jnp.int32),
        pltpu.SemaphoreType.DMA((7,)),
        pltpu.SemaphoreType.DMA((7,)),
        pltpu.SemaphoreType.DMA((7,)),
        pltpu.SemaphoreType.DMA((7,)),
    ],
)
def _sc_logit(tabs, src3, dst3, le8, loff16, out, maxout,
              idxs_v, idxd_v, bs, bd, bl, bo, mx_v, lo_v,
              sems, semd, seml, semo):
    c = lax.axis_index("c")
    s = lax.axis_index("s")
    wid = s * 2 + c
    base = wid * SLICE
    iota16 = lax.iota(jnp.int32, 16)
    pltpu.sync_copy(src3.at[wid], idxs_v)
    pltpu.sync_copy(dst3.at[wid], idxd_v)
    pltpu.sync_copy(loff16, lo_v)
    mx_v[...] = jnp.full((16,), -1e30, jnp.float32)
    neg = jnp.full((16,), -1e30, jnp.float32)
    for b in range(7):
        def pre(v, carry):
            rows = jnp.full((16,), v * 16, jnp.int32) + iota16
            for k in range(2, 8):
                plsc.store_scatter(
                    bo.at[b], [rows, jnp.full((16,), k, jnp.int32)], neg)
            return carry

        lax.fori_loop(0, 8, pre, 0)

    def gat(j, b):
        return [
            pltpu.make_async_copy(tabs.at[0].at[idxs_v.at[j]], bs.at[b],
                                  sems.at[b]),
            pltpu.make_async_copy(tabs.at[1].at[idxd_v.at[j]], bd.at[b],
                                  semd.at[b]),
            pltpu.make_async_copy(le8.at[pl.ds(base + j * 128, 128)],
                                  bl.at[b], seml.at[b]),
        ]

    def scat(j, b):
        return pltpu.make_async_copy(
            bo.at[b], out.at[pl.ds(base + j * 128, 128)], semo.at[b])

    scat.add = False
    lof = lo_v[...]

    def process(j, b):
        def rowgrp(v, carry2):
            rows = jnp.full((16,), v * 16, jnp.int32) + iota16
            for q in range(2):
                qf = jnp.full((16,), q, jnp.int32)
                z = (plsc.load_gather(bs.at[b], [rows, qf])
                     + plsc.load_gather(bd.at[b], [rows, qf])
                     + plsc.load_gather(bl.at[b], [rows, qf + lof]))
                z = jnp.maximum(z, 0.2 * z)
                plsc.store_scatter(bo.at[b], [rows, qf], z)
                mx_v[...] = jnp.maximum(mx_v[...], z)
            return carry2

        lax.fori_loop(0, 8, rowgrp, 0)

    _ring(CH, gat, scat, process=process)
    pltpu.sync_copy(mx_v, maxout.at[c, s])


ECH = 49


@_sc_kernel(
    out_type=jax.ShapeDtypeStruct((4, NP, 16), jnp.float32),
    scratch_types=[
        pltpu.VMEM((ECH, 128), jnp.int32),
        pltpu.VMEM((ECH, 128), jnp.int32),
        pltpu.VMEM((7, 128, 32), jnp.float32),
        pltpu.VMEM((7, 128, 8), jnp.float32),
        pltpu.VMEM((7, 128, 8), jnp.float32),
        pltpu.VMEM((7, 128, 16), jnp.float32),
        pltpu.VMEM((16,), jnp.float32),
        pltpu.VMEM_SHARED((NP, 16), jnp.float32),
        pltpu.SemaphoreType.DMA((7,)),
        pltpu.SemaphoreType.DMA((7,)),
        pltpu.SemaphoreType.DMA((7,)),
        pltpu.SemaphoreType.DMA((7,)),
    ],
)
def _sc_egat(tabw, src3, dst3, logit8, rden8, gm16, zeros16, out,
             idxs_v, idxd_v, gbufs, lbufs, rbufs, sbufs, gm_v, acc_sh,
             gsems, lsems, rsems, ssems):
    c = lax.axis_index("c")
    s = lax.axis_index("s")
    r0 = s * RB
    iota16 = lax.iota(jnp.int32, 16)
    iotahi = iota16 + 16
    pltpu.sync_copy(gm16, gm_v)
    gm = gm_v[...]
    c0f = jnp.zeros((16,), jnp.int32)
    c1f = c0f + 1

    def one_pass(p, carry0):
        cg = c * 2 + p
        pltpu.sync_copy(zeros16.at[pl.ds(r0, RB)], acc_sh.at[pl.ds(r0, RB)])
        plsc.subcore_barrier()

        def one_seg(seg, carry1):
            sl = s * 2 + seg // 4
            ch0 = (seg % 4) * ECH
            base = sl * SLICE + ch0 * 128
            pltpu.sync_copy(src3.at[sl, pl.ds(ch0, ECH)], idxs_v)
            pltpu.sync_copy(dst3.at[sl, pl.ds(ch0, ECH)], idxd_v)

            def gat(j, b):
                return [
                    pltpu.make_async_copy(
                        tabw.at[cg].at[idxs_v.at[j]], gbufs.at[b],
                        gsems.at[b]),
                    pltpu.make_async_copy(
                        logit8.at[pl.ds(base + j * 128, 128)], lbufs.at[b],
                        lsems.at[b]),
                    pltpu.make_async_copy(
                        rden8.at[idxd_v.at[j]], rbufs.at[b], rsems.at[b]),
                ]

            def scat(j, b):
                return pltpu.make_async_copy(
                    sbufs.at[b], acc_sh.at[idxd_v.at[j]], ssems.at[b])

            scat.add = True

            dnums = lax.GatherDimensionNumbers(
                offset_dims=(), collapsed_slice_dims=(0,),
                start_index_map=(0,))

            def bcast(vec, l):
                return lax.gather(
                    vec, jnp.full((16, 1), l, jnp.int32), dnums, (1,),
                    mode=lax.GatherScatterMode.PROMISE_IN_BOUNDS)

            def process(j, b):
                gbuf = gbufs.at[b]
                lbuf = lbufs.at[b]
                rbuf = rbufs.at[b]
                sbuf = sbufs.at[b]

                def rowgrp(v, carry2):
                    for vv2 in range(2):
                        v0 = v * 32 + vv2 * 16
                        rows = jnp.full((16,), v0, jnp.int32) + iota16
                        exw = []
                        for q in range(2):
                            qf = c0f + q
                            z = plsc.load_gather(lbuf, [rows, qf])
                            rd = plsc.load_gather(rbuf, [rows, qf])
                            exw.append(jnp.exp(z - gm) * rd)
                        for l in range(16):
                            rr = v0 + l
                            a = gbuf[rr, pl.ds(0, 16)]
                            bb = gbuf[rr, pl.ds(16, 16)]
                            sbuf[rr, pl.ds(0, 16)] = (
                                a * bcast(exw[0], l) + bb * bcast(exw[1], l))
                    return carry2

                lax.fori_loop(0, 4, rowgrp, 0)

            _ring(ECH, gat, scat, process=process)
            return carry1

        lax.fori_loop(0, 8, one_seg, 0)
        plsc.subcore_barrier()
        pltpu.sync_copy(acc_sh.at[pl.ds(r0, RB)],
                        out.at[cg, pl.ds(r0, RB)])
        plsc.subcore_barrier()
        return carry0

    lax.fori_loop(0, 2, one_pass, 0)


def _full(x):
    return pl.BlockSpec(x.shape, lambda i: (0,) * x.ndim)


def _t_edge(ea, We1, be1, ge1, bte1, We2, be2, ge2, bte2, ae1, ae2):
    def body(ea_r, We1_r, be1_r, ge1_r, bte1_r, We2_r, be2_r, ge2_r, bte2_r,
             ae1_r, ae2_r, le_r):
        x = ea_r[...]
        e1 = jnp.maximum((_mm(x, We1_r[...]) + be1_r[...]) * ge1_r[...]
                         + bte1_r[...], 0.0)
        e2 = jnp.maximum((_mm(e1, We2_r[...]) + be2_r[...]) * ge2_r[...]
                         + bte2_r[...], 0.0)
        le_r[...] = jnp.concatenate(
            [_mm(e2, ae1_r[...].T), _mm(e2, ae2_r[...].T),
             jnp.zeros((e2.shape[0], 4), jnp.float32)], axis=1)

    args = (ea, We1, be1, ge1, bte1, We2, be2, ge2, bte2, ae1, ae2)
    return pl.pallas_call(
        body,
        grid=(EG,),
        in_specs=[pl.BlockSpec((EB, NE), lambda i: (i, 0))]
        + [_full(a) for a in args[1:]],
        out_specs=pl.BlockSpec((EB, 8), lambda i: (i, 0)),
        out_shape=jax.ShapeDtypeStruct((EP, 8), jnp.float32),
    )(*args)


def _t_front(dxp, cxp, degp, Wc, bc, Wg0, bg0, Wgc1):
    def body(dx_r, cx_r, deg_r, Wc_r, bc_r, Wg0_r, bg0_r, Wgc1_r,
             xgdc_r, xs1_r):
        x_d = dx_r[...][:, 6:20]
        cx = cx_r[...]
        Wcv = Wc_r[...]
        bcv = bc_r[...]
        parts = [jnp.maximum(_mm(cx[:, 10 * g:10 * g + 10], Wcv) + bcv, 0.0)
                 for g in range(3)]
        xgdc = jnp.concatenate([x_d] + parts, axis=1)
        xgdc_r[...] = xgdc
        xg = jnp.maximum(_mm(xgdc, Wg0_r[...]) + bg0_r[...], 0.0)
        deg = deg_r[0, :, 0] + deg_r[1, :, 0] + 1.0
        dinv = lax.rsqrt(deg)[:, None]
        xs = _mm(xg, Wgc1_r[...]) * dinv
        for g in range(4):
            xs1_r[g] = xs[:, 16 * g:16 * g + 16]

    args = (dxp, cxp, degp, Wc, bc, Wg0, bg0, Wgc1)
    return pl.pallas_call(
        body,
        grid=(NG,),
        in_specs=[pl.BlockSpec((RB, 20), lambda i: (i, 0)),
                  pl.BlockSpec((RB, 30), lambda i: (i, 0)),
                  pl.BlockSpec((2, RB, 8), lambda i: (0, i, 0))]
        + [_full(a) for a in args[3:]],
        out_specs=[pl.BlockSpec((RB, 26), lambda i: (i, 0)),
                   pl.BlockSpec((4, RB, 16), lambda i: (0, i, 0))],
        out_shape=[jax.ShapeDtypeStruct((NP, 26), jnp.float32),
                   jax.ShapeDtypeStruct((4, NP, 16), jnp.float32)],
    )(*args)


def _t_gcn_mid(acc1, xs1, degp, bgc1, Wgc2):
    def body(acc_r, xs_r, deg_r, bgc1_r, Wgc2_r, xg0_r, xs2_r):
        deg = deg_r[0, :, 0] + deg_r[1, :, 0] + 1.0
        dinv = lax.rsqrt(deg)[:, None]
        tot = jnp.concatenate([acc_r[g] + xs_r[g] for g in range(4)],
                              axis=1)
        xg0 = jnp.maximum(tot * dinv + bgc1_r[...], 0.0)
        xg0_r[...] = xg0
        xs = _mm(xg0, Wgc2_r[...]) * dinv
        for g in range(4):
            xs2_r[g] = xs[:, 16 * g:16 * g + 16]

    args = (acc1, xs1, degp, bgc1, Wgc2)
    return pl.pallas_call(
        body,
        grid=(NG,),
        in_specs=[pl.BlockSpec((4, RB, 16), lambda i: (0, i, 0)),
                  pl.BlockSpec((4, RB, 16), lambda i: (0, i, 0)),
                  pl.BlockSpec((2, RB, 8), lambda i: (0, i, 0)),
                  _full(bgc1), _full(Wgc2)],
        out_specs=[pl.BlockSpec((RB, 64), lambda i: (i, 0)),
                   pl.BlockSpec((4, RB, 16), lambda i: (0, i, 0))],
        out_shape=[jax.ShapeDtypeStruct((NP, 64), jnp.float32),
                   jax.ShapeDtypeStruct((4, NP, 16), jnp.float32)],
    )(*args)


def _t_main(acc2, xs2, degp, xg0, xgdc, bgc2, Wr1, br1, gr1, btr1,
            Wr2, br2, gr2, btr2, wcr, bcr, Wf, bf, Ws0, bs0, Wa1, as1, ad1):
    def body(acc_r, xs_r, deg_r, xg0_r, xgdc_r, bgc2_r, Wr1_r, br1_r, gr1_r,
             btr1_r, Wr2_r, br2_r, gr2_r, btr2_r, wcr_r, bcr_r, Wf_r, bf_r,
             Ws0_r, bs0_r, Wa1_r, as1_r, ad1_r, hci_r, xw1_r, tab1_r):
        deg = deg_r[0, :, 0] + deg_r[1, :, 0] + 1.0
        dinv = lax.rsqrt(deg)[:, None]
        tot = jnp.concatenate([acc_r[g] + xs_r[g] for g in range(4)],
                              axis=1)
        xg1 = jnp.maximum(tot * dinv + bgc2_r[...], 0.0)
        x = jnp.concatenate([xgdc_r[...], xg0_r[...] + xg1], axis=1)
        hd = x
        hd = jnp.maximum((_mm(hd, Wr1_r[...]) + br1_r[...]) * gr1_r[...]
                         + btr1_r[...], 0.0) + hd
        hd = jnp.maximum((_mm(hd, Wr2_r[...]) + br2_r[...]) * gr2_r[...]
                         + btr2_r[...], 0.0) + hd
        wcr_v = wcr_r[...]
        bcr_v = bcr_r[...]
        xl = x
        for i in range(2):
            sv = _mm(xl, wcr_v[i][:, None])
            xl = x * sv + bcr_v[i] + xl
        hci_r[...] = jnp.maximum(_mm(hd + xl, Wf_r[...]) + bf_r[...], 0.0)
        xsi = jnp.maximum(_mm(x, Ws0_r[...]) + bs0_r[...], 0.0)
        Wa = Wa1_r[...]
        asv = as1_r[...]
        adv = ad1_r[...]
        scols, dcols = [], []
        xws = []
        for h in range(2):
            xw = _mm(xsi, Wa[h])
            xws.append(xw)
            scols.append(_mm(xw, asv[h][:, None]))
            dcols.append(_mm(xw, adv[h][:, None]))
        zw = jnp.zeros((xws[0].shape[0], 96), jnp.float32)
        for g in range(4):
            xw1_r[g] = jnp.concatenate(
                [xws[0][:, 16 * g:16 * g + 16],
                 xws[1][:, 16 * g:16 * g + 16], zw], axis=1)
        nrow = xsi.shape[0]
        rid = (pl.program_id(0) * nrow
               + lax.broadcasted_iota(jnp.int32, (nrow, 1), 0))
        zpad = jnp.zeros((nrow, 14), jnp.float32)
        tab1_r[0] = jnp.concatenate(scols + [zpad], axis=1)
        ld2 = jnp.where(rid < N, jnp.concatenate(dcols, axis=1), -4e29)
        tab1_r[1] = jnp.concatenate([ld2, zpad], axis=1)

    args = (acc2, xs2, degp, xg0, xgdc, bgc2, Wr1, br1, gr1, btr1,
            Wr2, br2, gr2, btr2, wcr, bcr, Wf, bf, Ws0, bs0, Wa1, as1, ad1)
    RBM = RB // 2
    return pl.pallas_call(
        body,
        grid=(NP // RBM,),
        in_specs=[pl.BlockSpec((4, RBM, 16), lambda i: (0, i, 0)),
                  pl.BlockSpec((4, RBM, 16), lambda i: (0, i, 0)),
                  pl.BlockSpec((2, RBM, 8), lambda i: (0, i, 0)),
                  pl.BlockSpec((RBM, 64), lambda i: (i, 0)),
                  pl.BlockSpec((RBM, 26), lambda i: (i, 0))]
        + [_full(a) for a in args[5:]],
        out_specs=[pl.BlockSpec((RBM, 64), lambda i: (i, 0)),
                   pl.BlockSpec((4, RBM, 128), lambda i: (0, i, 0)),
                   pl.BlockSpec((2, RBM, 16), lambda i: (0, i, 0))],
        out_shape=[jax.ShapeDtypeStruct((N, 64), jnp.float32),
                   jax.ShapeDtypeStruct((4, NP, 128), jnp.float32),
                   jax.ShapeDtypeStruct((2, NP, 16), jnp.float32)],
    )(*args)


def _t_rden(denp):
    def body(den_r, out_r):
        d0 = (den_r[0, :, 0] + den_r[1, :, 0] + 1e-16)[:, None]
        d1 = (den_r[0, :, 1] + den_r[1, :, 1] + 1e-16)[:, None]
        nrow = d0.shape[0]
        out_r[...] = jnp.concatenate(
            [1.0 / d0, 1.0 / d1, jnp.zeros((nrow, 6), jnp.float32)], axis=1)

    return pl.pallas_call(
        body,
        grid=(NG,),
        in_specs=[pl.BlockSpec((2, RB, 8), lambda i: (0, i, 0))],
        out_specs=pl.BlockSpec((RB, 8), lambda i: (i, 0)),
        out_shape=jax.ShapeDtypeStruct((NP, 8), jnp.float32),
    )(denp)


def _t_combine(num, bA, Wa2=None, as2=None, ad2=None, x_prev=None,
               make_tables=False, out_n=None):
    def body(*refs):
        if make_tables:
            (num_r, bA_r, Wa2_r, as2_r, ad2_r,
             xsi_r, xw2_r, tab2_r) = refs
        else:
            (num_r, bA_r, xp_r, hsi_r) = refs
        cols = [0.5 * num_r[cg] for cg in range(4)]
        x = jnp.maximum(jnp.concatenate(cols, axis=1) + bA_r[...], 0.0)
        if make_tables:
            xsi_r[...] = x
            Wa = Wa2_r[...]
            asv = as2_r[...]
            adv = ad2_r[...]
            scols, dcols = [], []
            xws = []
            for h in range(2):
                xw = _mm(x, Wa[h])
                xws.append(xw)
                scols.append(_mm(xw, asv[h][:, None]))
                dcols.append(_mm(xw, adv[h][:, None]))
            zw = jnp.zeros((xws[0].shape[0], 96), jnp.float32)
            for g in range(4):
                xw2_r[g] = jnp.concatenate(
                    [xws[0][:, 16 * g:16 * g + 16],
                     xws[1][:, 16 * g:16 * g + 16], zw], axis=1)
            nrow = x.shape[0]
            rid = (pl.program_id(0) * nrow
                   + lax.broadcasted_iota(jnp.int32, (nrow, 1), 0))
            zpad = jnp.zeros((nrow, 14), jnp.float32)
            tab2_r[0] = jnp.concatenate(scols + [zpad], axis=1)
            ld2 = jnp.where(rid < N, jnp.concatenate(dcols, axis=1), -4e29)
            tab2_r[1] = jnp.concatenate([ld2, zpad], axis=1)
        else:
            hsi_r[...] = x + xp_r[...]

    RBC = RB // 4
    base_specs = [pl.BlockSpec((4, RBC, 16), lambda i: (0, i, 0)),
                  _full(bA)]
    if make_tables:
        args = (num, bA, Wa2, as2, ad2)
        return pl.pallas_call(
            body,
            grid=(NP // RBC,),
            in_specs=base_specs + [_full(Wa2), _full(as2), _full(ad2)],
            out_specs=[pl.BlockSpec((RBC, 64), lambda i: (i, 0)),
                       pl.BlockSpec((4, RBC, 128), lambda i: (0, i, 0)),
                       pl.BlockSpec((2, RBC, 16), lambda i: (0, i, 0))],
            out_shape=[jax.ShapeDtypeStruct((NP, 64), jnp.float32),
                       jax.ShapeDtypeStruct((4, NP, 128), jnp.float32),
                       jax.ShapeDtypeStruct((2, NP, 16), jnp.float32)],
        )(*args)
    args = (num, bA, x_prev)
    return pl.pallas_call(
        body,
        grid=(NP // RBC,),
        in_specs=base_specs + [pl.BlockSpec((RBC, 64), lambda i: (i, 0))],
        out_specs=pl.BlockSpec((RBC, 64), lambda i: (i, 0)),
        out_shape=jax.ShapeDtypeStruct((out_n, 64), jnp.float32),
    )(*args)


def _t_head(h_ci, h_si, t, Wy0, by0, Wy1, by1, Wp0a, bp0a, Wp0b, bp0b,
            Wp1a, bp1a, Wp1b, bp1b, WpT, bpT):
    def body(hc_r, hs_r, t_r, Wy0_r, by0_r, Wy1_r, by1_r, Wp0a_r, bp0a_r,
             Wp0b_r, bp0b_r, Wp1a_r, bp1a_r, Wp1b_r, bp1b_r, WpT_r, bpT_r,
             py_r, pycf_r, py0_r, py1_r, pT_r):
        hc = hc_r[...]
        hs = hs_r[...]
        h = jnp.concatenate([hc, hs], axis=1)

        def smax(z):
            z = z - jnp.max(z, axis=1, keepdims=True)
            ez = jnp.exp(z)
            return ez / jnp.sum(ez, axis=1, keepdims=True)

        a0 = smax(_mm(h, Wy0_r[...]) + by0_r[...])
        a1 = smax(_mm(h, Wy1_r[...]) + by1_r[...])
        py0 = a0[:, :64] * hc + a0[:, 64:] * hs
        py1 = a1[:, :64] * hc + a1[:, 64:] * hs
        py0 = jax.nn.sigmoid(
            _mm(jnp.maximum(_mm(py0, Wp0a_r[...]) + bp0a_r[...], 0.0),
                Wp0b_r[...]) + bp0b_r[...])
        py1 = jax.nn.sigmoid(
            _mm(jnp.maximum(_mm(py1, Wp1a_r[...]) + bp1a_r[...], 0.0),
                Wp1b_r[...]) + bp1b_r[...])
        pT = jax.nn.sigmoid(_mm(hs, WpT_r[...]) + bpT_r[...])
        tv = t_r[...]
        py_r[...] = (1.0 - tv) * py0 + tv * py1
        pycf_r[...] = tv * py0 + (1.0 - tv) * py1
        py0_r[...] = py0
        py1_r[...] = py1
        pT_r[...] = pT

    args = (h_ci, h_si, t, Wy0, by0, Wy1, by1, Wp0a, bp0a, Wp0b, bp0b,
            Wp1a, bp1a, Wp1b, bp1b, WpT, bpT)
    HB = 2000
    o = pl.BlockSpec((HB, 1), lambda i: (i, 0))
    sd = jax.ShapeDtypeStruct((B, 1), jnp.float32)
    return pl.pallas_call(
        body,
        grid=(B // HB,),
        in_specs=[pl.BlockSpec((HB, 64), lambda i: (i, 0)),
                  pl.BlockSpec((HB, 64), lambda i: (i, 0)),
                  pl.BlockSpec((HB, 1), lambda i: (i, 0))]
        + [_full(a) for a in args[3:]],
        out_specs=[o, o, o, o, o],
        out_shape=[sd, sd, sd, sd, sd],
    )(*args)


def kernel(discrete_x, continous_x, edge_index, edge_attr, churn_date, t,
           Wc, bc, We1, be1, ge1, bte1, We2, be2, ge2, bte2,
           Wg0, bg0, Wgc1, bgc1, Wgc2, bgc2,
           Wr1, br1, gr1, btr1, Wr2, br2, gr2, btr2,
           wcr, bcr, Wf, bf, Ws0, bs0,
           Wa1, as1, ad1, ae1, bA1, Wa2, as2, ad2, ae2, bA2,
           Wy0, by0, Wy1, by1,
           Wp0a, bp0a, Wp0b, bp0b, Wp1a, bp1a, Wp1b, bp1b, WpT, bpT):
    f32 = jnp.float32
    src = edge_index[0].astype(jnp.int32)
    dst = edge_index[1].astype(jnp.int32)
    src_p = jnp.concatenate([src, jnp.zeros((EP - E,), jnp.int32)])
    dst_p = jnp.concatenate([dst, jnp.full((EP - E,), TRASH, jnp.int32)])
    src3 = src_p.reshape(32, CH, 128)
    dst3 = dst_p.reshape(32, CH, 128)
    src3q = (src_p * 4).reshape(32, CH, 128)
    zeros8 = jnp.zeros((NP, 8), f32)
    zeros16 = jnp.zeros((NP, 16), f32)
    ones8 = jnp.ones((128, 8), f32)
    loff1 = jnp.zeros((16,), jnp.int32)
    loff2 = jnp.full((16,), 2, jnp.int32)

    degp = _sc_degree(dst3, zeros8, ones8)
    le8 = _t_edge(edge_attr, We1, be1, ge1, bte1, We2, be2, ge2, bte2,
                  ae1, ae2)
    xgdc, xs1 = _t_front(discrete_x, continous_x, degp, Wc, bc, Wg0, bg0,
                         Wgc1)

    acc1 = _sc_gcn(xs1, src3, dst3, zeros16)
    xg0, xs2 = _t_gcn_mid(acc1, xs1, degp, bgc1, Wgc2)
    acc2 = _sc_gcn(xs2, src3, dst3, zeros16)

    h_ci, xw1, tabs1 = _t_main(
        acc2, xs2, degp, xg0, xgdc, bgc2, Wr1, br1, gr1, btr1,
        Wr2, br2, gr2, btr2, wcr, bcr, Wf, bf, Ws0, bs0, Wa1, as1, ad1)

    logit1, mx1 = _sc_logit(tabs1, src3, dst3, le8, loff1)
    gm16_1 = jnp.broadcast_to(jnp.max(mx1), (16,)).astype(f32)
    den1 = _sc_den(dst3, logit1, gm16_1, zeros8)
    rden1 = _t_rden(den1)
    xw1v = xw1.reshape(4, 4 * NP, 32)
    num1 = _sc_egat(xw1v, src3q, dst3, logit1, rden1, gm16_1, zeros16)
    x_si0, xw2, tabs2 = _t_combine(num1, bA1, Wa2=Wa2, as2=as2,
                                   ad2=ad2, make_tables=True)

    logit2, mx2 = _sc_logit(tabs2, src3, dst3, le8, loff2)
    gm16_2 = jnp.broadcast_to(jnp.max(mx2), (16,)).astype(f32)
    den2 = _sc_den(dst3, logit2, gm16_2, zeros8)
    rden2 = _t_rden(den2)
    xw2v = xw2.reshape(4, 4 * NP, 32)
    num2 = _sc_egat(xw2v, src3q, dst3, logit2, rden2, gm16_2, zeros16)
    h_si = _t_combine(num2, bA2, x_prev=x_si0, out_n=N)

    pred_y, pred_y_cf, py0, py1, pred_T = _t_head(
        h_ci, h_si, t, Wy0, by0, Wy1, by1, Wp0a, bp0a, Wp0b, bp0b,
        Wp1a, bp1a, Wp1b, bp1b, WpT, bpT)
    return (pred_y, pred_y_cf, py0, py1, pred_T, h_ci, h_si)
